# Initial kernel scaffold; baseline (speedup 1.0000x reference)
import jax
import jax.numpy as jnp
from jax.experimental import pallas as pl


def kernel(x, x_bond, pos, params, edge_index_intra, edge_index_inter, batch):
    raise NotImplementedError("write your pallas kernel here")


# speedup vs baseline: 1.6665x; 1.6665x over previous
import functools

import jax
import jax.numpy as jnp
from jax import lax
from jax.experimental import pallas as pl
from jax.experimental.pallas import tpu as pltpu
from jax.experimental.pallas import tpu_sc as plsc

N_NODES = 10000
E_EDGES = 160000
NODE_DIM = 35
BOND_DIM = 10
HID = 256
HALF = 128
D_COUNT = 64
N_GRAPHS = 64

NC = 2
NS = 16
L = 16


EPW = E_EDGES // (NC * NS)
_G = (EPW + L - 1) // L
_EPAD = _G * L


def _d2_body(px, py, pz, row_c, col_c, row_n, col_n, d2_c, d2_n,
             px_v, py_v, pz_v, ir_v, ic_v, out_v):
  c = lax.axis_index("c")
  s = lax.axis_index("s")
  w = s * NC + c
  pltpu.sync_copy(px, px_v)
  pltpu.sync_copy(py, py_v)
  pltpu.sync_copy(pz, pz_v)
  base = w * EPW
  for row_h, col_h, out_h in ((row_c, col_c, d2_c), (row_n, col_n, d2_n)):
    pltpu.sync_copy(row_h.at[pl.ds(base, EPW)], ir_v.at[pl.ds(0, EPW)])
    pltpu.sync_copy(col_h.at[pl.ds(base, EPW)], ic_v.at[pl.ds(0, EPW)])

    def grp(g, carry):
      sl = pl.ds(g * L, L)
      ri = jnp.clip(ir_v[sl], 0, N_NODES - 1)
      ci = jnp.clip(ic_v[sl], 0, N_NODES - 1)
      dx = plsc.load_gather(px_v, [ri]) - plsc.load_gather(px_v, [ci])
      dy = plsc.load_gather(py_v, [ri]) - plsc.load_gather(py_v, [ci])
      dz = plsc.load_gather(pz_v, [ri]) - plsc.load_gather(pz_v, [ci])
      out_v[sl] = dx * dx + dy * dy + dz * dz
      return carry

    lax.fori_loop(0, _G, grp, 0)
    pltpu.sync_copy(out_v.at[pl.ds(0, EPW)], out_h.at[pl.ds(base, EPW)])


def _edge_d2(px, py, pz, row_c, col_c, row_n, col_n):
  mesh = plsc.VectorSubcoreMesh(core_axis_name="c", subcore_axis_name="s",
                                num_cores=NC, num_subcores=NS)
  fn = pl.kernel(
      _d2_body,
      out_type=[jax.ShapeDtypeStruct((E_EDGES,), jnp.float32),
                jax.ShapeDtypeStruct((E_EDGES,), jnp.float32)],
      mesh=mesh,
      scratch_types=[
          pltpu.VMEM((N_NODES,), jnp.float32),
          pltpu.VMEM((N_NODES,), jnp.float32),
          pltpu.VMEM((N_NODES,), jnp.float32),
          pltpu.VMEM((_EPAD,), jnp.int32),
          pltpu.VMEM((_EPAD,), jnp.int32),
          pltpu.VMEM((_EPAD,), jnp.float32),
      ],
      compiler_params=pltpu.CompilerParams(needs_layout_passes=False),
  )
  return fn(px, py, pz, row_c, col_c, row_n, col_n)


KE = 80
EPT = E_EDGES // NS
NCHUNK = EPT // KE
N_PAD = 10240
RPT = N_PAD // NS
_ZREP = RPT // KE


def _msg_body(with_hb, *refs):
  if with_hb:
    (h0, h1, rad0, rad1, hb0, hb1, row, col, agg0, agg1,
     accum, ir_v, ic_v, rows_v, rad_v, hb_v) = refs
  else:
    (h0, h1, rad0, rad1, row, col, agg0, agg1,
     accum, ir_v, ic_v, rows_v, rad_v) = refs
    hb0 = hb1 = hb_v = None
  c = lax.axis_index("c")
  s = lax.axis_index("s")

  def zrow(i, carry):
    for j in range(HALF // L):
      rad_v[i, pl.ds(j * L, L)] = jnp.zeros((L,), jnp.float32)
    return carry

  lax.fori_loop(0, KE, zrow, 0)
  base_row = s * RPT
  for t in range(_ZREP):
    pltpu.sync_copy(rad_v, accum.at[pl.ds(base_row + t * KE, KE)])
  plsc.subcore_barrier()

  ebase = s * EPT

  def chunk(k, carry):
    off = ebase + k * KE
    pltpu.sync_copy(row.at[pl.ds(off, KE)], ir_v)
    pltpu.sync_copy(col.at[pl.ds(off, KE)], ic_v)

    @pl.when(c == 0)
    def _():
      pltpu.sync_copy(h0.at[ir_v], rows_v)
      pltpu.sync_copy(rad0.at[pl.ds(off, KE)], rad_v)
      if with_hb:
        pltpu.sync_copy(hb0.at[pl.ds(off, KE)], hb_v)

    @pl.when(c == 1)
    def _():
      pltpu.sync_copy(h1.at[ir_v], rows_v)
      pltpu.sync_copy(rad1.at[pl.ds(off, KE)], rad_v)
      if with_hb:
        pltpu.sync_copy(hb1.at[pl.ds(off, KE)], hb_v)

    def mrow(i, carry2):
      for j in range(HALF // L):
        sl = pl.ds(j * L, L)
        v = rows_v[i, sl] * rad_v[i, sl]
        if with_hb:
          v = v + hb_v[i, sl]
        rows_v[i, sl] = v
      return carry2

    lax.fori_loop(0, KE, mrow, 0)
    pltpu.sync_copy(rows_v, accum.at[ic_v], add=True)
    return carry

  lax.fori_loop(0, NCHUNK, chunk, 0)
  plsc.subcore_barrier()

  @pl.when(c == 0)
  def _():
    pltpu.sync_copy(accum.at[pl.ds(base_row, RPT)],
                    agg0.at[pl.ds(base_row, RPT)])

  @pl.when(c == 1)
  def _():
    pltpu.sync_copy(accum.at[pl.ds(base_row, RPT)],
                    agg1.at[pl.ds(base_row, RPT)])


def _gather_scatter(h0, h1, rad0, rad1, row, col, hb0=None, hb1=None):
  with_hb = hb0 is not None
  mesh = plsc.VectorSubcoreMesh(core_axis_name="c", subcore_axis_name="s",
                                num_cores=NC, num_subcores=NS)
  scratch = [
      pltpu.VMEM_SHARED((N_PAD, HALF), jnp.float32),
      pltpu.VMEM((KE,), jnp.int32),
      pltpu.VMEM((KE,), jnp.int32),
      pltpu.VMEM((KE, HALF), jnp.float32),
      pltpu.VMEM((KE, HALF), jnp.float32),
  ]
  if with_hb:
    scratch.append(pltpu.VMEM((KE, HALF), jnp.float32))
  fn = pl.kernel(
      functools.partial(_msg_body, with_hb),
      out_type=[jax.ShapeDtypeStruct((N_PAD, HALF), jnp.float32),
                jax.ShapeDtypeStruct((N_PAD, HALF), jnp.float32)],
      mesh=mesh,
      scratch_types=scratch,
      compiler_params=pltpu.CompilerParams(needs_layout_passes=False),
  )
  if with_hb:
    a0, a1 = fn(h0, h1, rad0, rad1, hb0, hb1, row, col)
  else:
    a0, a1 = fn(h0, h1, rad0, rad1, row, col)
  return a0[:N_NODES], a1[:N_NODES]


def _silu(z):
  return z * jax.nn.sigmoid(z)


def _node_embed_body(x_ref, w_ref, b_ref, o0_ref, o1_ref):
  h = _silu(jnp.dot(x_ref[...], w_ref[...],
                    preferred_element_type=jnp.float32) + b_ref[...])
  o0_ref[...] = h[:, :HALF]
  o1_ref[...] = h[:, HALF:]


def _node_embed(x, w, b):
  return pl.pallas_call(
      _node_embed_body,
      out_shape=[jax.ShapeDtypeStruct((N_NODES, HALF), jnp.float32),
                 jax.ShapeDtypeStruct((N_NODES, HALF), jnp.float32)],
  )(x, w, b.reshape(1, HID))


BE = 2000


def _bond_embed_body(xb_ref, w_ref, b_ref, o0_ref, o1_ref):
  h = _silu(jnp.dot(xb_ref[...], w_ref[...],
                    preferred_element_type=jnp.float32) + b_ref[...])
  o0_ref[...] = h[:, :HALF]
  o1_ref[...] = h[:, HALF:]


def _bond_embed(xb, w, b):
  nb = E_EDGES // BE
  return pl.pallas_call(
      _bond_embed_body,
      grid=(nb,),
      in_specs=[pl.BlockSpec((BE, BOND_DIM), lambda i: (i, 0)),
                pl.BlockSpec((BOND_DIM, HID), lambda i: (0, 0)),
                pl.BlockSpec((1, HID), lambda i: (0, 0))],
      out_specs=[pl.BlockSpec((BE, HALF), lambda i: (i, 0)),
                 pl.BlockSpec((BE, HALF), lambda i: (i, 0))],
      out_shape=[jax.ShapeDtypeStruct((E_EDGES, HALF), jnp.float32),
                 jax.ShapeDtypeStruct((E_EDGES, HALF), jnp.float32)],
  )(xb, w, b.reshape(1, HID))


def _rad_body(d_max, d2_ref, w_ref, b_ref, o0_ref, o1_ref):
  centers = (lax.broadcasted_iota(jnp.int32, (D_COUNT,), 0).astype(jnp.float32)
             * (d_max / (D_COUNT - 1)))
  width = d_max / D_COUNT
  d = jnp.sqrt(d2_ref[0, 0] + 1e-8)
  rbf = jnp.exp(-(((d[:, None] - centers[None, :]) / width) ** 2))
  z = jnp.dot(rbf, w_ref[...], preferred_element_type=jnp.float32) + b_ref[...]
  rad = _silu(z)
  o0_ref[...] = rad[:, :HALF]
  o1_ref[...] = rad[:, HALF:]


def _rad(d2, w, b, d_max):
  nb = E_EDGES // BE
  return pl.pallas_call(
      functools.partial(_rad_body, d_max),
      grid=(nb,),
      in_specs=[pl.BlockSpec((1, 1, BE), lambda i: (i, 0, 0)),
                pl.BlockSpec((D_COUNT, HID), lambda i: (0, 0)),
                pl.BlockSpec((1, HID), lambda i: (0, 0))],
      out_specs=[pl.BlockSpec((BE, HALF), lambda i: (i, 0)),
                 pl.BlockSpec((BE, HALF), lambda i: (i, 0))],
      out_shape=[jax.ShapeDtypeStruct((E_EDGES, HALF), jnp.float32),
                 jax.ShapeDtypeStruct((E_EDGES, HALF), jnp.float32)],
  )(d2.reshape(nb, 1, BE), w, b.reshape(1, HID))


BN_BLK = 400
BN_NB = N_NODES // BN_BLK


def _upd1_body(h0, h1, ac0, ac1, an0, an1, wc, bc, wn, bn,
               lc0, lc1, ln0, ln1, sc, qc, sn, qn):
  h = jnp.concatenate([h0[...], h1[...]], axis=1)
  ac = jnp.concatenate([ac0[...], ac1[...]], axis=1)
  an = jnp.concatenate([an0[...], an1[...]], axis=1)
  zc = jnp.dot(h + ac, wc[...], preferred_element_type=jnp.float32) + bc[...]
  zc = jnp.where(zc >= 0, zc, 0.01 * zc)
  zn = jnp.dot(h + an, wn[...], preferred_element_type=jnp.float32) + bn[...]
  zn = jnp.where(zn >= 0, zn, 0.01 * zn)
  lc0[...] = zc[:, :HALF]
  lc1[...] = zc[:, HALF:]
  ln0[...] = zn[:, :HALF]
  ln1[...] = zn[:, HALF:]
  sc[...] = jnp.sum(zc, axis=0, keepdims=True)[None]
  qc[...] = jnp.sum(zc * zc, axis=0, keepdims=True)[None]
  sn[...] = jnp.sum(zn, axis=0, keepdims=True)[None]
  qn[...] = jnp.sum(zn * zn, axis=0, keepdims=True)[None]


def _upd1(h0, h1, ac0, ac1, an0, an1, wc, bc, wn, bn):
  half_spec = pl.BlockSpec((BN_BLK, HALF), lambda i: (i, 0))
  wspec = pl.BlockSpec((HID, HID), lambda i: (0, 0))
  bspec = pl.BlockSpec((1, HID), lambda i: (0, 0))
  pspec = pl.BlockSpec((1, 1, HID), lambda i: (i, 0, 0))
  return pl.pallas_call(
      _upd1_body,
      grid=(BN_NB,),
      in_specs=[half_spec] * 6 + [wspec, bspec, wspec, bspec],
      out_specs=[half_spec] * 4 + [pspec] * 4,
      out_shape=[jax.ShapeDtypeStruct((N_NODES, HALF), jnp.float32)] * 4 +
                [jax.ShapeDtypeStruct((BN_NB, 1, HID), jnp.float32)] * 4,
  )(h0, h1, ac0, ac1, an0, an1, wc, bc.reshape(1, HID), wn, bn.reshape(1, HID))


def _upd2_body(lc0, lc1, ln0, ln1, sc, qc, sn, qn, gc, betac, gn, betan,
               o0, o1):
  inv_n = 1.0 / N_NODES
  mc = jnp.sum(sc[...], axis=0) * inv_n
  vc = jnp.sum(qc[...], axis=0) * inv_n - mc * mc
  mn = jnp.sum(sn[...], axis=0) * inv_n
  vn = jnp.sum(qn[...], axis=0) * inv_n - mn * mn
  rc = jax.lax.rsqrt(vc + 1e-5)
  rn = jax.lax.rsqrt(vn + 1e-5)
  zc = jnp.concatenate([lc0[...], lc1[...]], axis=1)
  zn = jnp.concatenate([ln0[...], ln1[...]], axis=1)
  hc = (zc - mc) * rc * gc[...] + betac[...]
  hn = (zn - mn) * rn * gn[...] + betan[...]
  h = hc + hn
  o0[...] = h[:, :HALF]
  o1[...] = h[:, HALF:]


def _upd2(lc0, lc1, ln0, ln1, sc, qc, sn, qn, gc, betac, gn, betan):
  half_spec = pl.BlockSpec((BN_BLK, HALF), lambda i: (i, 0))
  pspec = pl.BlockSpec((BN_NB, 1, HID), lambda i: (0, 0, 0))
  bspec = pl.BlockSpec((1, HID), lambda i: (0, 0))
  return pl.pallas_call(
      _upd2_body,
      grid=(BN_NB,),
      in_specs=[half_spec] * 4 + [pspec] * 4 + [bspec] * 4,
      out_specs=[half_spec] * 2,
      out_shape=[jax.ShapeDtypeStruct((N_NODES, HALF), jnp.float32)] * 2,
  )(lc0, lc1, ln0, ln1, sc, qc, sn, qn,
    gc.reshape(1, HID), betac.reshape(1, HID),
    gn.reshape(1, HID), betan.reshape(1, HID))


def _head_body(h0, h1, batch, fw0, fb0, fg0, fbeta0, fw1, fb1, fg1, fbeta1,
               fw2, fb2, fg2, fbeta2, wout, bout, o_ref):
  h = jnp.concatenate([h0[...], h1[...]], axis=1)
  b = batch[...]
  gid = jax.lax.broadcasted_iota(jnp.int32, (N_NODES, N_GRAPHS), 1)
  onehot = (b == gid).astype(jnp.float32)
  emb = jax.lax.dot_general(onehot, h, (((0,), (0,)), ((), ())),
                            preferred_element_type=jnp.float32)
  z = emb
  for w, bb, g, beta in ((fw0, fb0, fg0, fbeta0), (fw1, fb1, fg1, fbeta1),
                         (fw2, fb2, fg2, fbeta2)):
    z = jnp.dot(z, w[...], preferred_element_type=jnp.float32) + bb[...]
    z = jnp.where(z >= 0, z, 0.01 * z)
    m = jnp.mean(z, axis=0, keepdims=True)
    v = jnp.mean(z * z, axis=0, keepdims=True) - m * m
    z = (z - m) * jax.lax.rsqrt(v + 1e-5) * g[...] + beta[...]
  out = jnp.dot(z, wout[...], preferred_element_type=jnp.float32) + bout[...]
  o_ref[...] = out.reshape(1, N_GRAPHS)


def _head(h0, h1, batch, fc, wout, bout):
  args = [h0, h1, batch.reshape(N_NODES, 1).astype(jnp.int32)]
  for p in fc:
    args += [p['W'], p['b'].reshape(1, HID), p['g'].reshape(1, HID),
             p['beta'].reshape(1, HID)]
  args += [wout, bout.reshape(1, 1)]
  return pl.pallas_call(
      _head_body,
      out_shape=jax.ShapeDtypeStruct((1, N_GRAPHS), jnp.float32),
  )(*args)


def kernel(x, x_bond, pos, params, edge_index_intra, edge_index_inter, batch):
  row_c = edge_index_intra[0].astype(jnp.int32)
  col_c = edge_index_intra[1].astype(jnp.int32)
  row_n = edge_index_inter[0].astype(jnp.int32)
  col_n = edge_index_inter[1].astype(jnp.int32)
  px = pos[:, 0].astype(jnp.float32)
  py = pos[:, 1].astype(jnp.float32)
  pz = pos[:, 2].astype(jnp.float32)

  d2_c, d2_n = _edge_d2(px, py, pz, row_c, col_c, row_n, col_n)

  h0, h1 = _node_embed(x, params['W_node'], params['b_node'])
  hb0, hb1 = _bond_embed(x_bond, params['W_bond'], params['b_bond'])

  for p in params['gconv']:
    radc0, radc1 = _rad(d2_c, p['Wcc'], p['bcc'], 6.0)
    radn0, radn1 = _rad(d2_n, p['Wcn'], p['bcn'], 10.0)
    ac0, ac1 = _gather_scatter(h0, h1, radc0, radc1, row_c, col_c, hb0, hb1)
    an0, an1 = _gather_scatter(h0, h1, radn0, radn1, row_n, col_n)
    lc0, lc1, ln0, ln1, sc, qc, sn, qn = _upd1(
        h0, h1, ac0, ac1, an0, an1, p['Wnc'], p['bnc'], p['Wnn'], p['bnn'])
    h0, h1 = _upd2(lc0, lc1, ln0, ln1, sc, qc, sn, qn,
                   p['gc'], p['betac'], p['gn'], p['betan'])

  out = _head(h0, h1, batch, params['fc'], params['W_out'], params['b_out'])
  return out.reshape(-1)


# speedup vs baseline: 2.8543x; 1.7128x over previous
import functools

import jax
import jax.numpy as jnp
from jax import lax
from jax.experimental import pallas as pl
from jax.experimental.pallas import tpu as pltpu
from jax.experimental.pallas import tpu_sc as plsc

N_NODES = 10000
E_EDGES = 160000
NODE_DIM = 35
BOND_DIM = 10
HID = 256
HALF = 128
D_COUNT = 64
N_GRAPHS = 64

NC = 2
NS = 16
L = 16


EPW = E_EDGES // (NC * NS)
_G = (EPW + L - 1) // L
_EPAD = _G * L


def _d2_body(px, py, pz, row_c, col_c, row_n, col_n, d2_c, d2_n,
             px_v, py_v, pz_v, ir_v, ic_v, out_v):
  c = lax.axis_index("c")
  s = lax.axis_index("s")
  w = s * NC + c
  pltpu.sync_copy(px, px_v)
  pltpu.sync_copy(py, py_v)
  pltpu.sync_copy(pz, pz_v)
  base = w * EPW
  for row_h, col_h, out_h in ((row_c, col_c, d2_c), (row_n, col_n, d2_n)):
    pltpu.sync_copy(row_h.at[pl.ds(base, EPW)], ir_v.at[pl.ds(0, EPW)])
    pltpu.sync_copy(col_h.at[pl.ds(base, EPW)], ic_v.at[pl.ds(0, EPW)])

    def grp(g, carry):
      sl = pl.ds(g * L, L)
      ri = jnp.clip(ir_v[sl], 0, N_NODES - 1)
      ci = jnp.clip(ic_v[sl], 0, N_NODES - 1)
      dx = plsc.load_gather(px_v, [ri]) - plsc.load_gather(px_v, [ci])
      dy = plsc.load_gather(py_v, [ri]) - plsc.load_gather(py_v, [ci])
      dz = plsc.load_gather(pz_v, [ri]) - plsc.load_gather(pz_v, [ci])
      out_v[sl] = dx * dx + dy * dy + dz * dz
      return carry

    lax.fori_loop(0, _G, grp, 0)
    pltpu.sync_copy(out_v.at[pl.ds(0, EPW)], out_h.at[pl.ds(base, EPW)])


def _edge_d2(px, py, pz, row_c, col_c, row_n, col_n):
  mesh = plsc.VectorSubcoreMesh(core_axis_name="c", subcore_axis_name="s",
                                num_cores=NC, num_subcores=NS)
  fn = pl.kernel(
      _d2_body,
      out_type=[jax.ShapeDtypeStruct((E_EDGES,), jnp.float32),
                jax.ShapeDtypeStruct((E_EDGES,), jnp.float32)],
      mesh=mesh,
      scratch_types=[
          pltpu.VMEM((N_NODES,), jnp.float32),
          pltpu.VMEM((N_NODES,), jnp.float32),
          pltpu.VMEM((N_NODES,), jnp.float32),
          pltpu.VMEM((_EPAD,), jnp.int32),
          pltpu.VMEM((_EPAD,), jnp.int32),
          pltpu.VMEM((_EPAD,), jnp.float32),
      ],
      compiler_params=pltpu.CompilerParams(needs_layout_passes=False),
  )
  return fn(px, py, pz, row_c, col_c, row_n, col_n)


KE = 40
EPT = E_EDGES // NS
NCHUNK = EPT // KE
N_PAD = 10240
RPT = N_PAD // NS
_ZREP = RPT // KE


def _zero_accum(accum, buf, s):
  def zrow(i, carry):
    for j in range(HALF // L):
      buf[i, pl.ds(j * L, L)] = jnp.zeros((L,), jnp.float32)
    return carry

  lax.fori_loop(0, KE, zrow, 0)
  base_row = s * RPT
  for t in range(_ZREP):
    pltpu.sync_copy(buf, accum.at[pl.ds(base_row + t * KE, KE)])


def _msg_body(h0, h1, rad0, rad1, row, col, agg0, agg1,
              accum, row_s, rows_a, rows_b, rad_a, rad_b, col_a, col_b, sem):
  c = lax.axis_index("c")
  s = lax.axis_index("s")
  _zero_accum(accum, rad_a, s)
  plsc.subcore_barrier()

  ebase = s * EPT
  pltpu.sync_copy(row.at[pl.ds(ebase, EPT)], row_s)

  h_t = (h0, h1)
  rad_t = (rad0, rad1)

  def load(k, rows_v, rad_v, col_v, half):
    off = ebase + k * KE
    return [pltpu.async_copy(h_t[half].at[row_s.at[pl.ds(k * KE, KE)]],
                             rows_v, sem),
            pltpu.async_copy(rad_t[half].at[pl.ds(off, KE)], rad_v, sem),
            pltpu.async_copy(col.at[pl.ds(off, KE)], col_v, sem)]

  def process(rows_v, rad_v, col_v):
    def mrow(i, carry2):
      for ii in range(2):
        for j in range(HALF // L):
          sl = pl.ds(j * L, L)
          rows_v[2 * i + ii, sl] = rows_v[2 * i + ii, sl] * rad_v[2 * i + ii, sl]
      return carry2

    lax.fori_loop(0, KE // 2, mrow, 0)
    pltpu.sync_copy(rows_v, accum.at[col_v], add=True)

  for half in range(NC):

    @pl.when(c == half)
    def _():
      for d in load(0, rows_a, rad_a, col_a, half):
        d.wait()

      def pair(g, carry):
        k = 2 * g
        da = load(k + 1, rows_b, rad_b, col_b, half)
        process(rows_a, rad_a, col_a)
        for d in da:
          d.wait()
        db = load(k + 2, rows_a, rad_a, col_a, half)
        process(rows_b, rad_b, col_b)
        for d in db:
          d.wait()
        return carry

      lax.fori_loop(0, NCHUNK // 2 - 1, pair, 0)
      da = load(NCHUNK - 1, rows_b, rad_b, col_b, half)
      process(rows_a, rad_a, col_a)
      for d in da:
        d.wait()
      process(rows_b, rad_b, col_b)

  plsc.subcore_barrier()
  base_row = s * RPT

  @pl.when(c == 0)
  def _():
    pltpu.sync_copy(accum.at[pl.ds(base_row, RPT)],
                    agg0.at[pl.ds(base_row, RPT)])

  @pl.when(c == 1)
  def _():
    pltpu.sync_copy(accum.at[pl.ds(base_row, RPT)],
                    agg1.at[pl.ds(base_row, RPT)])


def _gather_scatter(h0, h1, rad0, rad1, row, col):
  mesh = plsc.VectorSubcoreMesh(core_axis_name="c", subcore_axis_name="s",
                                num_cores=NC, num_subcores=NS)
  scratch = [
      pltpu.VMEM_SHARED((N_PAD, HALF), jnp.float32),
      pltpu.VMEM((EPT,), jnp.int32),
      pltpu.VMEM((KE, HALF), jnp.float32),
      pltpu.VMEM((KE, HALF), jnp.float32),
      pltpu.VMEM((KE, HALF), jnp.float32),
      pltpu.VMEM((KE, HALF), jnp.float32),
      pltpu.VMEM((KE,), jnp.int32),
      pltpu.VMEM((KE,), jnp.int32),
      pltpu.SemaphoreType.DMA,
  ]
  fn = pl.kernel(
      _msg_body,
      out_type=[jax.ShapeDtypeStruct((N_PAD, HALF), jnp.float32),
                jax.ShapeDtypeStruct((N_PAD, HALF), jnp.float32)],
      mesh=mesh,
      scratch_types=scratch,
      compiler_params=pltpu.CompilerParams(needs_layout_passes=False),
  )
  a0, a1 = fn(h0, h1, rad0, rad1, row, col)
  return a0[:N_NODES], a1[:N_NODES]


def _hb_body(hb0, hb1, col, agg0, agg1,
             accum, buf_a, buf_b, col_a, col_b, sem):
  c = lax.axis_index("c")
  s = lax.axis_index("s")
  _zero_accum(accum, buf_a, s)
  plsc.subcore_barrier()
  hb_t = (hb0, hb1)
  ebase = s * EPT

  for half in range(NC):

    @pl.when(c == half)
    def _():
      def load(k, buf_v, col_v):
        off = ebase + k * KE
        return [pltpu.async_copy(hb_t[half].at[pl.ds(off, KE)], buf_v, sem),
                pltpu.async_copy(col.at[pl.ds(off, KE)], col_v, sem)]

      for d in load(0, buf_a, col_a):
        d.wait()

      def pair(g, carry):
        k = 2 * g
        da = load(k + 1, buf_b, col_b)
        pltpu.sync_copy(buf_a, accum.at[col_a], add=True)
        for d in da:
          d.wait()
        db = load(k + 2, buf_a, col_a)
        pltpu.sync_copy(buf_b, accum.at[col_b], add=True)
        for d in db:
          d.wait()
        return carry

      lax.fori_loop(0, NCHUNK // 2 - 1, pair, 0)
      da = load(NCHUNK - 1, buf_b, col_b)
      pltpu.sync_copy(buf_a, accum.at[col_a], add=True)
      for d in da:
        d.wait()
      pltpu.sync_copy(buf_b, accum.at[col_b], add=True)

  plsc.subcore_barrier()

  base_row = s * RPT

  @pl.when(c == 0)
  def _():
    pltpu.sync_copy(accum.at[pl.ds(base_row, RPT)],
                    agg0.at[pl.ds(base_row, RPT)])

  @pl.when(c == 1)
  def _():
    pltpu.sync_copy(accum.at[pl.ds(base_row, RPT)],
                    agg1.at[pl.ds(base_row, RPT)])


def _hb_scatter(hb0, hb1, col):
  mesh = plsc.VectorSubcoreMesh(core_axis_name="c", subcore_axis_name="s",
                                num_cores=NC, num_subcores=NS)
  fn = pl.kernel(
      _hb_body,
      out_type=[jax.ShapeDtypeStruct((N_PAD, HALF), jnp.float32),
                jax.ShapeDtypeStruct((N_PAD, HALF), jnp.float32)],
      mesh=mesh,
      scratch_types=[
          pltpu.VMEM_SHARED((N_PAD, HALF), jnp.float32),
          pltpu.VMEM((KE, HALF), jnp.float32),
          pltpu.VMEM((KE, HALF), jnp.float32),
          pltpu.VMEM((KE,), jnp.int32),
          pltpu.VMEM((KE,), jnp.int32),
          pltpu.SemaphoreType.DMA,
      ],
      compiler_params=pltpu.CompilerParams(needs_layout_passes=False),
  )
  a0, a1 = fn(hb0, hb1, col)
  return a0[:N_NODES], a1[:N_NODES]


def _silu(z):
  return z * jax.nn.sigmoid(z)


def _node_embed_body(x_ref, w_ref, b_ref, o0_ref, o1_ref):
  h = _silu(jnp.dot(x_ref[...], w_ref[...],
                    preferred_element_type=jnp.float32) + b_ref[...])
  o0_ref[...] = h[:, :HALF]
  o1_ref[...] = h[:, HALF:]


def _node_embed(x, w, b):
  return pl.pallas_call(
      _node_embed_body,
      out_shape=[jax.ShapeDtypeStruct((N_NODES, HALF), jnp.float32),
                 jax.ShapeDtypeStruct((N_NODES, HALF), jnp.float32)],
  )(x, w, b.reshape(1, HID))


BE = 2000


def _bond_embed_body(xb_ref, w_ref, b_ref, o0_ref, o1_ref):
  h = _silu(jnp.dot(xb_ref[...], w_ref[...],
                    preferred_element_type=jnp.float32) + b_ref[...])
  o0_ref[...] = h[:, :HALF]
  o1_ref[...] = h[:, HALF:]


def _bond_embed(xb, w, b):
  nb = E_EDGES // BE
  return pl.pallas_call(
      _bond_embed_body,
      grid=(nb,),
      in_specs=[pl.BlockSpec((BE, BOND_DIM), lambda i: (i, 0)),
                pl.BlockSpec((BOND_DIM, HID), lambda i: (0, 0)),
                pl.BlockSpec((1, HID), lambda i: (0, 0))],
      out_specs=[pl.BlockSpec((BE, HALF), lambda i: (i, 0)),
                 pl.BlockSpec((BE, HALF), lambda i: (i, 0))],
      out_shape=[jax.ShapeDtypeStruct((E_EDGES, HALF), jnp.float32),
                 jax.ShapeDtypeStruct((E_EDGES, HALF), jnp.float32)],
  )(xb, w, b.reshape(1, HID))


def _rad_body(d_max, d2_ref, w_ref, b_ref, o0_ref, o1_ref):
  centers = (lax.broadcasted_iota(jnp.int32, (D_COUNT,), 0).astype(jnp.float32)
             * (d_max / (D_COUNT - 1)))
  width = d_max / D_COUNT
  d = jnp.sqrt(d2_ref[0, 0] + 1e-8)
  rbf = jnp.exp(-(((d[:, None] - centers[None, :]) / width) ** 2))
  z = jnp.dot(rbf, w_ref[...], preferred_element_type=jnp.float32) + b_ref[...]
  rad = _silu(z)
  o0_ref[...] = rad[:, :HALF]
  o1_ref[...] = rad[:, HALF:]


def _rad(d2, w, b, d_max):
  nb = E_EDGES // BE
  return pl.pallas_call(
      functools.partial(_rad_body, d_max),
      grid=(nb,),
      in_specs=[pl.BlockSpec((1, 1, BE), lambda i: (i, 0, 0)),
                pl.BlockSpec((D_COUNT, HID), lambda i: (0, 0)),
                pl.BlockSpec((1, HID), lambda i: (0, 0))],
      out_specs=[pl.BlockSpec((BE, HALF), lambda i: (i, 0)),
                 pl.BlockSpec((BE, HALF), lambda i: (i, 0))],
      out_shape=[jax.ShapeDtypeStruct((E_EDGES, HALF), jnp.float32),
                 jax.ShapeDtypeStruct((E_EDGES, HALF), jnp.float32)],
  )(d2.reshape(nb, 1, BE), w, b.reshape(1, HID))


BN_BLK = 400
BN_NB = N_NODES // BN_BLK


def _upd1_body(h0, h1, ac0, ac1, hba0, hba1, an0, an1, wc, bc, wn, bn,
               lc0, lc1, ln0, ln1, sc, qc, sn, qn):
  h = jnp.concatenate([h0[...], h1[...]], axis=1)
  ac = (jnp.concatenate([ac0[...], ac1[...]], axis=1) +
        jnp.concatenate([hba0[...], hba1[...]], axis=1))
  an = jnp.concatenate([an0[...], an1[...]], axis=1)
  zc = jnp.dot(h + ac, wc[...], preferred_element_type=jnp.float32) + bc[...]
  zc = jnp.where(zc >= 0, zc, 0.01 * zc)
  zn = jnp.dot(h + an, wn[...], preferred_element_type=jnp.float32) + bn[...]
  zn = jnp.where(zn >= 0, zn, 0.01 * zn)
  lc0[...] = zc[:, :HALF]
  lc1[...] = zc[:, HALF:]
  ln0[...] = zn[:, :HALF]
  ln1[...] = zn[:, HALF:]
  sc[...] = jnp.sum(zc, axis=0, keepdims=True)[None]
  qc[...] = jnp.sum(zc * zc, axis=0, keepdims=True)[None]
  sn[...] = jnp.sum(zn, axis=0, keepdims=True)[None]
  qn[...] = jnp.sum(zn * zn, axis=0, keepdims=True)[None]


def _upd1(h0, h1, ac0, ac1, hba0, hba1, an0, an1, wc, bc, wn, bn):
  half_spec = pl.BlockSpec((BN_BLK, HALF), lambda i: (i, 0))
  wspec = pl.BlockSpec((HID, HID), lambda i: (0, 0))
  bspec = pl.BlockSpec((1, HID), lambda i: (0, 0))
  pspec = pl.BlockSpec((1, 1, HID), lambda i: (i, 0, 0))
  return pl.pallas_call(
      _upd1_body,
      grid=(BN_NB,),
      in_specs=[half_spec] * 8 + [wspec, bspec, wspec, bspec],
      out_specs=[half_spec] * 4 + [pspec] * 4,
      out_shape=[jax.ShapeDtypeStruct((N_NODES, HALF), jnp.float32)] * 4 +
                [jax.ShapeDtypeStruct((BN_NB, 1, HID), jnp.float32)] * 4,
  )(h0, h1, ac0, ac1, hba0, hba1, an0, an1,
    wc, bc.reshape(1, HID), wn, bn.reshape(1, HID))


def _upd2_body(lc0, lc1, ln0, ln1, sc, qc, sn, qn, gc, betac, gn, betan,
               o0, o1):
  inv_n = 1.0 / N_NODES
  mc = jnp.sum(sc[...], axis=0) * inv_n
  vc = jnp.sum(qc[...], axis=0) * inv_n - mc * mc
  mn = jnp.sum(sn[...], axis=0) * inv_n
  vn = jnp.sum(qn[...], axis=0) * inv_n - mn * mn
  rc = jax.lax.rsqrt(vc + 1e-5)
  rn = jax.lax.rsqrt(vn + 1e-5)
  zc = jnp.concatenate([lc0[...], lc1[...]], axis=1)
  zn = jnp.concatenate([ln0[...], ln1[...]], axis=1)
  hc = (zc - mc) * rc * gc[...] + betac[...]
  hn = (zn - mn) * rn * gn[...] + betan[...]
  h = hc + hn
  o0[...] = h[:, :HALF]
  o1[...] = h[:, HALF:]


def _upd2(lc0, lc1, ln0, ln1, sc, qc, sn, qn, gc, betac, gn, betan):
  half_spec = pl.BlockSpec((BN_BLK, HALF), lambda i: (i, 0))
  pspec = pl.BlockSpec((BN_NB, 1, HID), lambda i: (0, 0, 0))
  bspec = pl.BlockSpec((1, HID), lambda i: (0, 0))
  return pl.pallas_call(
      _upd2_body,
      grid=(BN_NB,),
      in_specs=[half_spec] * 4 + [pspec] * 4 + [bspec] * 4,
      out_specs=[half_spec] * 2,
      out_shape=[jax.ShapeDtypeStruct((N_NODES, HALF), jnp.float32)] * 2,
  )(lc0, lc1, ln0, ln1, sc, qc, sn, qn,
    gc.reshape(1, HID), betac.reshape(1, HID),
    gn.reshape(1, HID), betan.reshape(1, HID))


def _head_body(h0, h1, batch, fw0, fb0, fg0, fbeta0, fw1, fb1, fg1, fbeta1,
               fw2, fb2, fg2, fbeta2, wout, bout, o_ref):
  h = jnp.concatenate([h0[...], h1[...]], axis=1)
  b = batch[...]
  gid = jax.lax.broadcasted_iota(jnp.int32, (N_NODES, N_GRAPHS), 1)
  onehot = (b == gid).astype(jnp.float32)
  emb = jax.lax.dot_general(onehot, h, (((0,), (0,)), ((), ())),
                            preferred_element_type=jnp.float32)
  z = emb
  for w, bb, g, beta in ((fw0, fb0, fg0, fbeta0), (fw1, fb1, fg1, fbeta1),
                         (fw2, fb2, fg2, fbeta2)):
    z = jnp.dot(z, w[...], preferred_element_type=jnp.float32) + bb[...]
    z = jnp.where(z >= 0, z, 0.01 * z)
    m = jnp.mean(z, axis=0, keepdims=True)
    v = jnp.mean(z * z, axis=0, keepdims=True) - m * m
    z = (z - m) * jax.lax.rsqrt(v + 1e-5) * g[...] + beta[...]
  out = jnp.dot(z, wout[...], preferred_element_type=jnp.float32) + bout[...]
  o_ref[...] = out.reshape(1, N_GRAPHS)


def _head(h0, h1, batch, fc, wout, bout):
  args = [h0, h1, batch.reshape(N_NODES, 1).astype(jnp.int32)]
  for p in fc:
    args += [p['W'], p['b'].reshape(1, HID), p['g'].reshape(1, HID),
             p['beta'].reshape(1, HID)]
  args += [wout, bout.reshape(1, 1)]
  return pl.pallas_call(
      _head_body,
      out_shape=jax.ShapeDtypeStruct((1, N_GRAPHS), jnp.float32),
  )(*args)


def kernel(x, x_bond, pos, params, edge_index_intra, edge_index_inter, batch):
  row_c = edge_index_intra[0].astype(jnp.int32)
  col_c = edge_index_intra[1].astype(jnp.int32)
  row_n = edge_index_inter[0].astype(jnp.int32)
  col_n = edge_index_inter[1].astype(jnp.int32)
  px = pos[:, 0].astype(jnp.float32)
  py = pos[:, 1].astype(jnp.float32)
  pz = pos[:, 2].astype(jnp.float32)

  d2_c, d2_n = _edge_d2(px, py, pz, row_c, col_c, row_n, col_n)

  h0, h1 = _node_embed(x, params['W_node'], params['b_node'])
  hb0, hb1 = _bond_embed(x_bond, params['W_bond'], params['b_bond'])
  hba0, hba1 = _hb_scatter(hb0, hb1, col_c)

  for p in params['gconv']:
    radc0, radc1 = _rad(d2_c, p['Wcc'], p['bcc'], 6.0)
    radn0, radn1 = _rad(d2_n, p['Wcn'], p['bcn'], 10.0)
    ac0, ac1 = _gather_scatter(h0, h1, radc0, radc1, row_c, col_c)
    an0, an1 = _gather_scatter(h0, h1, radn0, radn1, row_n, col_n)
    lc0, lc1, ln0, ln1, sc, qc, sn, qn = _upd1(
        h0, h1, ac0, ac1, hba0, hba1, an0, an1,
        p['Wnc'], p['bnc'], p['Wnn'], p['bnn'])
    h0, h1 = _upd2(lc0, lc1, ln0, ln1, sc, qc, sn, qn,
                   p['gc'], p['betac'], p['gn'], p['betan'])

  out = _head(h0, h1, batch, params['fc'], params['W_out'], params['b_out'])
  return out.reshape(-1)


# speedup vs baseline: 3.2856x; 1.1511x over previous
import functools

import jax
import jax.numpy as jnp
from jax import lax
from jax.experimental import pallas as pl
from jax.experimental.pallas import tpu as pltpu
from jax.experimental.pallas import tpu_sc as plsc

N_NODES = 10000
E_EDGES = 160000
NODE_DIM = 35
BOND_DIM = 10
HID = 256
HALF = 128
D_COUNT = 64
N_GRAPHS = 64

NC = 2
NS = 16
L = 16


EPW = E_EDGES // (NC * NS)
_G = (EPW + L - 1) // L
_EPAD = _G * L


def _d2_body(px, py, pz, row_c, col_c, row_n, col_n, d2_c, d2_n,
             px_v, py_v, pz_v, ir_v, ic_v, out_v):
  c = lax.axis_index("c")
  s = lax.axis_index("s")
  w = s * NC + c
  pltpu.sync_copy(px, px_v)
  pltpu.sync_copy(py, py_v)
  pltpu.sync_copy(pz, pz_v)
  base = w * EPW
  for row_h, col_h, out_h in ((row_c, col_c, d2_c), (row_n, col_n, d2_n)):
    pltpu.sync_copy(row_h.at[pl.ds(base, EPW)], ir_v.at[pl.ds(0, EPW)])
    pltpu.sync_copy(col_h.at[pl.ds(base, EPW)], ic_v.at[pl.ds(0, EPW)])

    def grp(g, carry):
      sl = pl.ds(g * L, L)
      ri = jnp.clip(ir_v[sl], 0, N_NODES - 1)
      ci = jnp.clip(ic_v[sl], 0, N_NODES - 1)
      dx = plsc.load_gather(px_v, [ri]) - plsc.load_gather(px_v, [ci])
      dy = plsc.load_gather(py_v, [ri]) - plsc.load_gather(py_v, [ci])
      dz = plsc.load_gather(pz_v, [ri]) - plsc.load_gather(pz_v, [ci])
      out_v[sl] = dx * dx + dy * dy + dz * dz
      return carry

    lax.fori_loop(0, _G, grp, 0)
    pltpu.sync_copy(out_v.at[pl.ds(0, EPW)], out_h.at[pl.ds(base, EPW)])


def _edge_d2(px, py, pz, row_c, col_c, row_n, col_n):
  mesh = plsc.VectorSubcoreMesh(core_axis_name="c", subcore_axis_name="s",
                                num_cores=NC, num_subcores=NS)
  fn = pl.kernel(
      _d2_body,
      out_type=[jax.ShapeDtypeStruct((E_EDGES,), jnp.float32),
                jax.ShapeDtypeStruct((E_EDGES,), jnp.float32)],
      mesh=mesh,
      scratch_types=[
          pltpu.VMEM((N_NODES,), jnp.float32),
          pltpu.VMEM((N_NODES,), jnp.float32),
          pltpu.VMEM((N_NODES,), jnp.float32),
          pltpu.VMEM((_EPAD,), jnp.int32),
          pltpu.VMEM((_EPAD,), jnp.int32),
          pltpu.VMEM((_EPAD,), jnp.float32),
      ],
      compiler_params=pltpu.CompilerParams(needs_layout_passes=False),
  )
  return fn(px, py, pz, row_c, col_c, row_n, col_n)


KE = 40
EPT = E_EDGES // NS
NCHUNK = EPT // KE
N_PAD = 10240
RPT = N_PAD // NS
_ZREP = RPT // KE


def _zero_accum(accum, buf, s):
  def zrow(i, carry):
    for j in range(HALF // L):
      buf[i, pl.ds(j * L, L)] = jnp.zeros((L,), jnp.float32)
    return carry

  lax.fori_loop(0, KE, zrow, 0)
  base_row = s * RPT
  for t in range(_ZREP):
    pltpu.sync_copy(buf, accum.at[pl.ds(base_row + t * KE, KE)])


def _msg_body(h0, h1, rad0, rad1, row, col, agg0, agg1,
              accum, row_s, rows0, rows1, rows2, radv0, radv1, radv2,
              colv0, colv1, colv2, semd0, semd1, semd2, sems0, sems1, sems2):
  c = lax.axis_index("c")
  s = lax.axis_index("s")
  rows_r = (rows0, rows1, rows2)
  rad_r = (radv0, radv1, radv2)
  col_r = (colv0, colv1, colv2)
  semd = (semd0, semd1, semd2)
  sems = (sems0, sems1, sems2)
  _zero_accum(accum, radv0, s)
  plsc.subcore_barrier()

  ebase = s * EPT
  pltpu.sync_copy(row.at[pl.ds(ebase, EPT)], row_s)

  h_t = (h0, h1)
  rad_t = (rad0, rad1)

  for half in range(NC):

    @pl.when(c == half)
    def _():
      def descs(k, r):
        off = ebase + k * KE
        return [
            pltpu.make_async_copy(h_t[half].at[row_s.at[pl.ds(k * KE, KE)]],
                                  rows_r[r], semd[r]),
            pltpu.make_async_copy(rad_t[half].at[pl.ds(off, KE)],
                                  rad_r[r], semd[r]),
            pltpu.make_async_copy(col.at[pl.ds(off, KE)], col_r[r], semd[r]),
        ]

      def issue_load(k, r):
        for d in descs(k, r):
          d.start()

      def wait_load(k, r):
        for d in descs(k, r):
          d.wait()

      def mult(r):
        rows_v, rad_v = rows_r[r], rad_r[r]

        def mrow(i, carry2):
          for ii in range(2):
            for j in range(HALF // L):
              sl = pl.ds(j * L, L)
              rows_v[2 * i + ii, sl] = (rows_v[2 * i + ii, sl] *
                                        rad_v[2 * i + ii, sl])
          return carry2

        lax.fori_loop(0, KE // 2, mrow, 0)

      def scatter_issue(r):
        pltpu.async_copy(rows_r[r], accum.at[col_r[r]], sems[r], add=True)

      def scatter_wait(r):
        pltpu.make_async_copy(rows_r[r], accum.at[col_r[r]], sems[r]).wait()

      issue_load(0, 0)
      wait_load(0, 0)
      issue_load(1, 1)
      mult(0)
      scatter_issue(0)
      wait_load(1, 1)
      issue_load(2, 2)

      def tri(t, carry):
        k = 3 * t + 1
        mult(1)
        scatter_issue(1)
        wait_load(k + 1, 2)
        scatter_wait(0)
        issue_load(k + 2, 0)
        mult(2)
        scatter_issue(2)
        scatter_wait(1)

        @pl.when(k + 3 < NCHUNK)
        def _():
          issue_load(k + 3, 1)

        wait_load(k + 2, 0)
        mult(0)
        scatter_issue(0)
        scatter_wait(2)

        @pl.when(k + 4 < NCHUNK)
        def _():
          issue_load(k + 4, 2)

        @pl.when(k + 3 < NCHUNK)
        def _():
          wait_load(k + 3, 1)

        return carry

      lax.fori_loop(0, (NCHUNK - 1) // 3, tri, 0)
      scatter_wait(0)

  plsc.subcore_barrier()
  base_row = s * RPT

  @pl.when(c == 0)
  def _():
    pltpu.sync_copy(accum.at[pl.ds(base_row, RPT)],
                    agg0.at[pl.ds(base_row, RPT)])

  @pl.when(c == 1)
  def _():
    pltpu.sync_copy(accum.at[pl.ds(base_row, RPT)],
                    agg1.at[pl.ds(base_row, RPT)])


def _gather_scatter(h0, h1, rad0, rad1, row, col):
  mesh = plsc.VectorSubcoreMesh(core_axis_name="c", subcore_axis_name="s",
                                num_cores=NC, num_subcores=NS)
  scratch = (
      [pltpu.VMEM_SHARED((N_PAD, HALF), jnp.float32),
       pltpu.VMEM((EPT,), jnp.int32)] +
      [pltpu.VMEM((KE, HALF), jnp.float32)] * 3 +
      [pltpu.VMEM((KE, HALF), jnp.float32)] * 3 +
      [pltpu.VMEM((KE,), jnp.int32)] * 3 +
      [pltpu.SemaphoreType.DMA] * 6
  )
  fn = pl.kernel(
      _msg_body,
      out_type=[jax.ShapeDtypeStruct((N_PAD, HALF), jnp.float32),
                jax.ShapeDtypeStruct((N_PAD, HALF), jnp.float32)],
      mesh=mesh,
      scratch_types=scratch,
      compiler_params=pltpu.CompilerParams(needs_layout_passes=False),
  )
  a0, a1 = fn(h0, h1, rad0, rad1, row, col)
  return a0[:N_NODES], a1[:N_NODES]


def _hb_body(hb0, hb1, col, agg0, agg1,
             accum, buf_a, buf_b, col_a, col_b, sem):
  c = lax.axis_index("c")
  s = lax.axis_index("s")
  _zero_accum(accum, buf_a, s)
  plsc.subcore_barrier()
  hb_t = (hb0, hb1)
  ebase = s * EPT

  for half in range(NC):

    @pl.when(c == half)
    def _():
      def load(k, buf_v, col_v):
        off = ebase + k * KE
        return [pltpu.async_copy(hb_t[half].at[pl.ds(off, KE)], buf_v, sem),
                pltpu.async_copy(col.at[pl.ds(off, KE)], col_v, sem)]

      for d in load(0, buf_a, col_a):
        d.wait()

      def pair(g, carry):
        k = 2 * g
        da = load(k + 1, buf_b, col_b)
        pltpu.sync_copy(buf_a, accum.at[col_a], add=True)
        for d in da:
          d.wait()
        db = load(k + 2, buf_a, col_a)
        pltpu.sync_copy(buf_b, accum.at[col_b], add=True)
        for d in db:
          d.wait()
        return carry

      lax.fori_loop(0, NCHUNK // 2 - 1, pair, 0)
      da = load(NCHUNK - 1, buf_b, col_b)
      pltpu.sync_copy(buf_a, accum.at[col_a], add=True)
      for d in da:
        d.wait()
      pltpu.sync_copy(buf_b, accum.at[col_b], add=True)

  plsc.subcore_barrier()

  base_row = s * RPT

  @pl.when(c == 0)
  def _():
    pltpu.sync_copy(accum.at[pl.ds(base_row, RPT)],
                    agg0.at[pl.ds(base_row, RPT)])

  @pl.when(c == 1)
  def _():
    pltpu.sync_copy(accum.at[pl.ds(base_row, RPT)],
                    agg1.at[pl.ds(base_row, RPT)])


def _hb_scatter(hb0, hb1, col):
  mesh = plsc.VectorSubcoreMesh(core_axis_name="c", subcore_axis_name="s",
                                num_cores=NC, num_subcores=NS)
  fn = pl.kernel(
      _hb_body,
      out_type=[jax.ShapeDtypeStruct((N_PAD, HALF), jnp.float32),
                jax.ShapeDtypeStruct((N_PAD, HALF), jnp.float32)],
      mesh=mesh,
      scratch_types=[
          pltpu.VMEM_SHARED((N_PAD, HALF), jnp.float32),
          pltpu.VMEM((KE, HALF), jnp.float32),
          pltpu.VMEM((KE, HALF), jnp.float32),
          pltpu.VMEM((KE,), jnp.int32),
          pltpu.VMEM((KE,), jnp.int32),
          pltpu.SemaphoreType.DMA,
      ],
      compiler_params=pltpu.CompilerParams(needs_layout_passes=False),
  )
  a0, a1 = fn(hb0, hb1, col)
  return a0[:N_NODES], a1[:N_NODES]


def _silu(z):
  return z * jax.nn.sigmoid(z)


def _node_embed_body(x_ref, w_ref, b_ref, o0_ref, o1_ref):
  h = _silu(jnp.dot(x_ref[...], w_ref[...],
                    preferred_element_type=jnp.float32) + b_ref[...])
  o0_ref[...] = h[:, :HALF]
  o1_ref[...] = h[:, HALF:]


def _node_embed(x, w, b):
  return pl.pallas_call(
      _node_embed_body,
      out_shape=[jax.ShapeDtypeStruct((N_NODES, HALF), jnp.float32),
                 jax.ShapeDtypeStruct((N_NODES, HALF), jnp.float32)],
  )(x, w, b.reshape(1, HID))


BE = 2000


def _bond_embed_body(xb_ref, w_ref, b_ref, o0_ref, o1_ref):
  h = _silu(jnp.dot(xb_ref[...], w_ref[...],
                    preferred_element_type=jnp.float32) + b_ref[...])
  o0_ref[...] = h[:, :HALF]
  o1_ref[...] = h[:, HALF:]


def _bond_embed(xb, w, b):
  nb = E_EDGES // BE
  return pl.pallas_call(
      _bond_embed_body,
      grid=(nb,),
      in_specs=[pl.BlockSpec((BE, BOND_DIM), lambda i: (i, 0)),
                pl.BlockSpec((BOND_DIM, HID), lambda i: (0, 0)),
                pl.BlockSpec((1, HID), lambda i: (0, 0))],
      out_specs=[pl.BlockSpec((BE, HALF), lambda i: (i, 0)),
                 pl.BlockSpec((BE, HALF), lambda i: (i, 0))],
      out_shape=[jax.ShapeDtypeStruct((E_EDGES, HALF), jnp.float32),
                 jax.ShapeDtypeStruct((E_EDGES, HALF), jnp.float32)],
  )(xb, w, b.reshape(1, HID))


def _rad_body(d_max, d2_ref, w_ref, b_ref, o0_ref, o1_ref):
  centers = (lax.broadcasted_iota(jnp.int32, (D_COUNT,), 0).astype(jnp.float32)
             * (d_max / (D_COUNT - 1)))
  width = d_max / D_COUNT
  d = jnp.sqrt(d2_ref[0, 0] + 1e-8)
  rbf = jnp.exp(-(((d[:, None] - centers[None, :]) / width) ** 2))
  z = jnp.dot(rbf, w_ref[...], preferred_element_type=jnp.float32) + b_ref[...]
  rad = _silu(z)
  o0_ref[...] = rad[:, :HALF]
  o1_ref[...] = rad[:, HALF:]


def _rad(d2, w, b, d_max):
  nb = E_EDGES // BE
  return pl.pallas_call(
      functools.partial(_rad_body, d_max),
      grid=(nb,),
      in_specs=[pl.BlockSpec((1, 1, BE), lambda i: (i, 0, 0)),
                pl.BlockSpec((D_COUNT, HID), lambda i: (0, 0)),
                pl.BlockSpec((1, HID), lambda i: (0, 0))],
      out_specs=[pl.BlockSpec((BE, HALF), lambda i: (i, 0)),
                 pl.BlockSpec((BE, HALF), lambda i: (i, 0))],
      out_shape=[jax.ShapeDtypeStruct((E_EDGES, HALF), jnp.float32),
                 jax.ShapeDtypeStruct((E_EDGES, HALF), jnp.float32)],
  )(d2.reshape(nb, 1, BE), w, b.reshape(1, HID))


BN_BLK = 400
BN_NB = N_NODES // BN_BLK


def _upd1_body(h0, h1, ac0, ac1, hba0, hba1, an0, an1, wc, bc, wn, bn,
               lc0, lc1, ln0, ln1, sc, qc, sn, qn):
  h = jnp.concatenate([h0[...], h1[...]], axis=1)
  ac = (jnp.concatenate([ac0[...], ac1[...]], axis=1) +
        jnp.concatenate([hba0[...], hba1[...]], axis=1))
  an = jnp.concatenate([an0[...], an1[...]], axis=1)
  zc = jnp.dot(h + ac, wc[...], preferred_element_type=jnp.float32) + bc[...]
  zc = jnp.where(zc >= 0, zc, 0.01 * zc)
  zn = jnp.dot(h + an, wn[...], preferred_element_type=jnp.float32) + bn[...]
  zn = jnp.where(zn >= 0, zn, 0.01 * zn)
  lc0[...] = zc[:, :HALF]
  lc1[...] = zc[:, HALF:]
  ln0[...] = zn[:, :HALF]
  ln1[...] = zn[:, HALF:]
  sc[...] = jnp.sum(zc, axis=0, keepdims=True)[None]
  qc[...] = jnp.sum(zc * zc, axis=0, keepdims=True)[None]
  sn[...] = jnp.sum(zn, axis=0, keepdims=True)[None]
  qn[...] = jnp.sum(zn * zn, axis=0, keepdims=True)[None]


def _upd1(h0, h1, ac0, ac1, hba0, hba1, an0, an1, wc, bc, wn, bn):
  half_spec = pl.BlockSpec((BN_BLK, HALF), lambda i: (i, 0))
  wspec = pl.BlockSpec((HID, HID), lambda i: (0, 0))
  bspec = pl.BlockSpec((1, HID), lambda i: (0, 0))
  pspec = pl.BlockSpec((1, 1, HID), lambda i: (i, 0, 0))
  return pl.pallas_call(
      _upd1_body,
      grid=(BN_NB,),
      in_specs=[half_spec] * 8 + [wspec, bspec, wspec, bspec],
      out_specs=[half_spec] * 4 + [pspec] * 4,
      out_shape=[jax.ShapeDtypeStruct((N_NODES, HALF), jnp.float32)] * 4 +
                [jax.ShapeDtypeStruct((BN_NB, 1, HID), jnp.float32)] * 4,
  )(h0, h1, ac0, ac1, hba0, hba1, an0, an1,
    wc, bc.reshape(1, HID), wn, bn.reshape(1, HID))


def _upd2_body(lc0, lc1, ln0, ln1, sc, qc, sn, qn, gc, betac, gn, betan,
               o0, o1):
  inv_n = 1.0 / N_NODES
  mc = jnp.sum(sc[...], axis=0) * inv_n
  vc = jnp.sum(qc[...], axis=0) * inv_n - mc * mc
  mn = jnp.sum(sn[...], axis=0) * inv_n
  vn = jnp.sum(qn[...], axis=0) * inv_n - mn * mn
  rc = jax.lax.rsqrt(vc + 1e-5)
  rn = jax.lax.rsqrt(vn + 1e-5)
  zc = jnp.concatenate([lc0[...], lc1[...]], axis=1)
  zn = jnp.concatenate([ln0[...], ln1[...]], axis=1)
  hc = (zc - mc) * rc * gc[...] + betac[...]
  hn = (zn - mn) * rn * gn[...] + betan[...]
  h = hc + hn
  o0[...] = h[:, :HALF]
  o1[...] = h[:, HALF:]


def _upd2(lc0, lc1, ln0, ln1, sc, qc, sn, qn, gc, betac, gn, betan):
  half_spec = pl.BlockSpec((BN_BLK, HALF), lambda i: (i, 0))
  pspec = pl.BlockSpec((BN_NB, 1, HID), lambda i: (0, 0, 0))
  bspec = pl.BlockSpec((1, HID), lambda i: (0, 0))
  return pl.pallas_call(
      _upd2_body,
      grid=(BN_NB,),
      in_specs=[half_spec] * 4 + [pspec] * 4 + [bspec] * 4,
      out_specs=[half_spec] * 2,
      out_shape=[jax.ShapeDtypeStruct((N_NODES, HALF), jnp.float32)] * 2,
  )(lc0, lc1, ln0, ln1, sc, qc, sn, qn,
    gc.reshape(1, HID), betac.reshape(1, HID),
    gn.reshape(1, HID), betan.reshape(1, HID))


def _head_body(h0, h1, batch, fw0, fb0, fg0, fbeta0, fw1, fb1, fg1, fbeta1,
               fw2, fb2, fg2, fbeta2, wout, bout, o_ref):
  h = jnp.concatenate([h0[...], h1[...]], axis=1)
  b = batch[...]
  gid = jax.lax.broadcasted_iota(jnp.int32, (N_NODES, N_GRAPHS), 1)
  onehot = (b == gid).astype(jnp.float32)
  emb = jax.lax.dot_general(onehot, h, (((0,), (0,)), ((), ())),
                            preferred_element_type=jnp.float32)
  z = emb
  for w, bb, g, beta in ((fw0, fb0, fg0, fbeta0), (fw1, fb1, fg1, fbeta1),
                         (fw2, fb2, fg2, fbeta2)):
    z = jnp.dot(z, w[...], preferred_element_type=jnp.float32) + bb[...]
    z = jnp.where(z >= 0, z, 0.01 * z)
    m = jnp.mean(z, axis=0, keepdims=True)
    v = jnp.mean(z * z, axis=0, keepdims=True) - m * m
    z = (z - m) * jax.lax.rsqrt(v + 1e-5) * g[...] + beta[...]
  out = jnp.dot(z, wout[...], preferred_element_type=jnp.float32) + bout[...]
  o_ref[...] = out.reshape(1, N_GRAPHS)


def _head(h0, h1, batch, fc, wout, bout):
  args = [h0, h1, batch.reshape(N_NODES, 1).astype(jnp.int32)]
  for p in fc:
    args += [p['W'], p['b'].reshape(1, HID), p['g'].reshape(1, HID),
             p['beta'].reshape(1, HID)]
  args += [wout, bout.reshape(1, 1)]
  return pl.pallas_call(
      _head_body,
      out_shape=jax.ShapeDtypeStruct((1, N_GRAPHS), jnp.float32),
  )(*args)


def kernel(x, x_bond, pos, params, edge_index_intra, edge_index_inter, batch):
  row_c = edge_index_intra[0].astype(jnp.int32)
  col_c = edge_index_intra[1].astype(jnp.int32)
  row_n = edge_index_inter[0].astype(jnp.int32)
  col_n = edge_index_inter[1].astype(jnp.int32)
  px = pos[:, 0].astype(jnp.float32)
  py = pos[:, 1].astype(jnp.float32)
  pz = pos[:, 2].astype(jnp.float32)

  d2_c, d2_n = _edge_d2(px, py, pz, row_c, col_c, row_n, col_n)

  h0, h1 = _node_embed(x, params['W_node'], params['b_node'])
  hb0, hb1 = _bond_embed(x_bond, params['W_bond'], params['b_bond'])
  hba0, hba1 = _hb_scatter(hb0, hb1, col_c)

  for p in params['gconv']:
    radc0, radc1 = _rad(d2_c, p['Wcc'], p['bcc'], 6.0)
    radn0, radn1 = _rad(d2_n, p['Wcn'], p['bcn'], 10.0)
    ac0, ac1 = _gather_scatter(h0, h1, radc0, radc1, row_c, col_c)
    an0, an1 = _gather_scatter(h0, h1, radn0, radn1, row_n, col_n)
    lc0, lc1, ln0, ln1, sc, qc, sn, qn = _upd1(
        h0, h1, ac0, ac1, hba0, hba1, an0, an1,
        p['Wnc'], p['bnc'], p['Wnn'], p['bnn'])
    h0, h1 = _upd2(lc0, lc1, ln0, ln1, sc, qc, sn, qn,
                   p['gc'], p['betac'], p['gn'], p['betan'])

  out = _head(h0, h1, batch, params['fc'], params['W_out'], params['b_out'])
  return out.reshape(-1)


# speedup vs baseline: 3.2877x; 1.0006x over previous
import functools

import jax
import jax.numpy as jnp
from jax import lax
from jax.experimental import pallas as pl
from jax.experimental.pallas import tpu as pltpu
from jax.experimental.pallas import tpu_sc as plsc

N_NODES = 10000
E_EDGES = 160000
NODE_DIM = 35
BOND_DIM = 10
HID = 256
HALF = 128
D_COUNT = 64
N_GRAPHS = 64

NC = 2
NS = 16
L = 16


EPW = E_EDGES // (NC * NS)
_G = (EPW + L - 1) // L
_EPAD = _G * L


def _d2_body(px, py, pz, row_c, col_c, row_n, col_n, d2_c, d2_n,
             px_v, py_v, pz_v, ir_v, ic_v, out_v):
  c = lax.axis_index("c")
  s = lax.axis_index("s")
  w = s * NC + c
  pltpu.sync_copy(px, px_v)
  pltpu.sync_copy(py, py_v)
  pltpu.sync_copy(pz, pz_v)
  base = w * EPW
  for row_h, col_h, out_h in ((row_c, col_c, d2_c), (row_n, col_n, d2_n)):
    pltpu.sync_copy(row_h.at[pl.ds(base, EPW)], ir_v.at[pl.ds(0, EPW)])
    pltpu.sync_copy(col_h.at[pl.ds(base, EPW)], ic_v.at[pl.ds(0, EPW)])

    def grp(g, carry):
      sl = pl.ds(g * L, L)
      ri = jnp.clip(ir_v[sl], 0, N_NODES - 1)
      ci = jnp.clip(ic_v[sl], 0, N_NODES - 1)
      dx = plsc.load_gather(px_v, [ri]) - plsc.load_gather(px_v, [ci])
      dy = plsc.load_gather(py_v, [ri]) - plsc.load_gather(py_v, [ci])
      dz = plsc.load_gather(pz_v, [ri]) - plsc.load_gather(pz_v, [ci])
      out_v[sl] = dx * dx + dy * dy + dz * dz
      return carry

    lax.fori_loop(0, _G, grp, 0)
    pltpu.sync_copy(out_v.at[pl.ds(0, EPW)], out_h.at[pl.ds(base, EPW)])


def _edge_d2(px, py, pz, row_c, col_c, row_n, col_n):
  mesh = plsc.VectorSubcoreMesh(core_axis_name="c", subcore_axis_name="s",
                                num_cores=NC, num_subcores=NS)
  fn = pl.kernel(
      _d2_body,
      out_type=[jax.ShapeDtypeStruct((E_EDGES,), jnp.float32),
                jax.ShapeDtypeStruct((E_EDGES,), jnp.float32)],
      mesh=mesh,
      scratch_types=[
          pltpu.VMEM((N_NODES,), jnp.float32),
          pltpu.VMEM((N_NODES,), jnp.float32),
          pltpu.VMEM((N_NODES,), jnp.float32),
          pltpu.VMEM((_EPAD,), jnp.int32),
          pltpu.VMEM((_EPAD,), jnp.int32),
          pltpu.VMEM((_EPAD,), jnp.float32),
      ],
      compiler_params=pltpu.CompilerParams(needs_layout_passes=False),
  )
  return fn(px, py, pz, row_c, col_c, row_n, col_n)


KE = 40
EPT = E_EDGES // NS
NCHUNK = EPT // KE
N_PAD = 10240
RPT = N_PAD // NS
_ZREP = RPT // KE


def _zero_accum(accum, buf, s):
  def zrow(i, carry):
    for j in range(HALF // L):
      buf[i, pl.ds(j * L, L)] = jnp.zeros((L,), jnp.float32)
    return carry

  lax.fori_loop(0, KE, zrow, 0)
  base_row = s * RPT
  for t in range(_ZREP):
    pltpu.sync_copy(buf, accum.at[pl.ds(base_row + t * KE, KE)])


def _msg_body(h0, h1, rad0, rad1, row, col, agg0, agg1,
              accum, row_s, rows0, rows1, rows2, radv0, radv1, radv2,
              colv0, colv1, colv2, semd0, semd1, semd2, sems0, sems1, sems2):
  c = lax.axis_index("c")
  s = lax.axis_index("s")
  rows_r = (rows0, rows1, rows2)
  rad_r = (radv0, radv1, radv2)
  col_r = (colv0, colv1, colv2)
  semd = (semd0, semd1, semd2)
  sems = (sems0, sems1, sems2)
  _zero_accum(accum, radv0, s)
  plsc.subcore_barrier()

  ebase = s * EPT
  pltpu.sync_copy(row.at[pl.ds(ebase, EPT)], row_s)

  h_t = (h0, h1)
  rad_t = (rad0, rad1)

  for half in range(NC):

    @pl.when(c == half)
    def _():
      def descs(k, r):
        off = ebase + k * KE
        return [
            pltpu.make_async_copy(h_t[half].at[row_s.at[pl.ds(k * KE, KE)]],
                                  rows_r[r], semd[r]),
            pltpu.make_async_copy(rad_t[half].at[pl.ds(off, KE)],
                                  rad_r[r], semd[r]),
            pltpu.make_async_copy(col.at[pl.ds(off, KE)], col_r[r], semd[r]),
        ]

      def issue_load(k, r):
        for d in descs(k, r):
          d.start()

      def wait_load(k, r):
        for d in descs(k, r):
          d.wait()

      def mult(r):
        rows_v, rad_v = rows_r[r], rad_r[r]

        def mrow(i, carry2):
          for ii in range(2):
            for j in range(HALF // L):
              sl = pl.ds(j * L, L)
              rows_v[2 * i + ii, sl] = (rows_v[2 * i + ii, sl] *
                                        rad_v[2 * i + ii, sl])
          return carry2

        lax.fori_loop(0, KE // 2, mrow, 0)

      def scatter_issue(r):
        pltpu.async_copy(rows_r[r], accum.at[col_r[r]], sems[r], add=True)

      def scatter_wait(r):
        pltpu.make_async_copy(rows_r[r], accum.at[col_r[r]], sems[r]).wait()

      issue_load(0, 0)
      wait_load(0, 0)
      issue_load(1, 1)
      mult(0)
      scatter_issue(0)
      wait_load(1, 1)
      issue_load(2, 2)

      def tri(t, carry):
        k = 3 * t + 1
        mult(1)
        scatter_issue(1)
        wait_load(k + 1, 2)
        scatter_wait(0)
        issue_load(k + 2, 0)
        mult(2)
        scatter_issue(2)
        scatter_wait(1)

        @pl.when(k + 3 < NCHUNK)
        def _():
          issue_load(k + 3, 1)

        wait_load(k + 2, 0)
        mult(0)
        scatter_issue(0)
        scatter_wait(2)

        @pl.when(k + 4 < NCHUNK)
        def _():
          issue_load(k + 4, 2)

        @pl.when(k + 3 < NCHUNK)
        def _():
          wait_load(k + 3, 1)

        return carry

      lax.fori_loop(0, (NCHUNK - 1) // 3, tri, 0)
      scatter_wait(0)

  plsc.subcore_barrier()
  base_row = s * RPT

  @pl.when(c == 0)
  def _():
    pltpu.sync_copy(accum.at[pl.ds(base_row, RPT)],
                    agg0.at[pl.ds(base_row, RPT)])

  @pl.when(c == 1)
  def _():
    pltpu.sync_copy(accum.at[pl.ds(base_row, RPT)],
                    agg1.at[pl.ds(base_row, RPT)])


def _gather_scatter(h0, h1, rad0, rad1, row, col):
  mesh = plsc.VectorSubcoreMesh(core_axis_name="c", subcore_axis_name="s",
                                num_cores=NC, num_subcores=NS)
  scratch = (
      [pltpu.VMEM_SHARED((N_PAD, HALF), jnp.float32),
       pltpu.VMEM((EPT,), jnp.int32)] +
      [pltpu.VMEM((KE, HALF), jnp.float32)] * 3 +
      [pltpu.VMEM((KE, HALF), jnp.float32)] * 3 +
      [pltpu.VMEM((KE,), jnp.int32)] * 3 +
      [pltpu.SemaphoreType.DMA] * 6
  )
  fn = pl.kernel(
      _msg_body,
      out_type=[jax.ShapeDtypeStruct((N_PAD, HALF), jnp.float32),
                jax.ShapeDtypeStruct((N_PAD, HALF), jnp.float32)],
      mesh=mesh,
      scratch_types=scratch,
      compiler_params=pltpu.CompilerParams(needs_layout_passes=False),
  )
  a0, a1 = fn(h0, h1, rad0, rad1, row, col)
  return a0[:N_NODES], a1[:N_NODES]


def _hb_body(hb0, hb1, col, agg0, agg1,
             accum, buf_a, buf_b, col_a, col_b, sem):
  c = lax.axis_index("c")
  s = lax.axis_index("s")
  _zero_accum(accum, buf_a, s)
  plsc.subcore_barrier()
  hb_t = (hb0, hb1)
  ebase = s * EPT

  for half in range(NC):

    @pl.when(c == half)
    def _():
      def load(k, buf_v, col_v):
        off = ebase + k * KE
        return [pltpu.async_copy(hb_t[half].at[pl.ds(off, KE)], buf_v, sem),
                pltpu.async_copy(col.at[pl.ds(off, KE)], col_v, sem)]

      for d in load(0, buf_a, col_a):
        d.wait()

      def pair(g, carry):
        k = 2 * g
        da = load(k + 1, buf_b, col_b)
        pltpu.sync_copy(buf_a, accum.at[col_a], add=True)
        for d in da:
          d.wait()
        db = load(k + 2, buf_a, col_a)
        pltpu.sync_copy(buf_b, accum.at[col_b], add=True)
        for d in db:
          d.wait()
        return carry

      lax.fori_loop(0, NCHUNK // 2 - 1, pair, 0)
      da = load(NCHUNK - 1, buf_b, col_b)
      pltpu.sync_copy(buf_a, accum.at[col_a], add=True)
      for d in da:
        d.wait()
      pltpu.sync_copy(buf_b, accum.at[col_b], add=True)

  plsc.subcore_barrier()

  base_row = s * RPT

  @pl.when(c == 0)
  def _():
    pltpu.sync_copy(accum.at[pl.ds(base_row, RPT)],
                    agg0.at[pl.ds(base_row, RPT)])

  @pl.when(c == 1)
  def _():
    pltpu.sync_copy(accum.at[pl.ds(base_row, RPT)],
                    agg1.at[pl.ds(base_row, RPT)])


def _hb_scatter(hb0, hb1, col):
  mesh = plsc.VectorSubcoreMesh(core_axis_name="c", subcore_axis_name="s",
                                num_cores=NC, num_subcores=NS)
  fn = pl.kernel(
      _hb_body,
      out_type=[jax.ShapeDtypeStruct((N_PAD, HALF), jnp.float32),
                jax.ShapeDtypeStruct((N_PAD, HALF), jnp.float32)],
      mesh=mesh,
      scratch_types=[
          pltpu.VMEM_SHARED((N_PAD, HALF), jnp.float32),
          pltpu.VMEM((KE, HALF), jnp.float32),
          pltpu.VMEM((KE, HALF), jnp.float32),
          pltpu.VMEM((KE,), jnp.int32),
          pltpu.VMEM((KE,), jnp.int32),
          pltpu.SemaphoreType.DMA,
      ],
      compiler_params=pltpu.CompilerParams(needs_layout_passes=False),
  )
  a0, a1 = fn(hb0, hb1, col)
  return a0[:N_NODES], a1[:N_NODES]


def _silu(z):
  return z * jax.nn.sigmoid(z)


def _node_embed_body(x_ref, w_ref, b_ref, o0_ref, o1_ref):
  h = _silu(jnp.dot(x_ref[...], w_ref[...],
                    preferred_element_type=jnp.float32) + b_ref[...])
  o0_ref[...] = h[:, :HALF]
  o1_ref[...] = h[:, HALF:]


def _node_embed(x, w, b):
  return pl.pallas_call(
      _node_embed_body,
      out_shape=[jax.ShapeDtypeStruct((N_NODES, HALF), jnp.float32),
                 jax.ShapeDtypeStruct((N_NODES, HALF), jnp.float32)],
  )(x, w, b.reshape(1, HID))


BE = 2000


def _bond_embed_body(xb_ref, w_ref, b_ref, o0_ref, o1_ref):
  h = _silu(jnp.dot(xb_ref[...], w_ref[...],
                    preferred_element_type=jnp.float32) + b_ref[...])
  o0_ref[...] = h[:, :HALF]
  o1_ref[...] = h[:, HALF:]


def _bond_embed(xb, w, b):
  nb = E_EDGES // BE
  return pl.pallas_call(
      _bond_embed_body,
      grid=(nb,),
      in_specs=[pl.BlockSpec((BE, BOND_DIM), lambda i: (i, 0)),
                pl.BlockSpec((BOND_DIM, HID), lambda i: (0, 0)),
                pl.BlockSpec((1, HID), lambda i: (0, 0))],
      out_specs=[pl.BlockSpec((BE, HALF), lambda i: (i, 0)),
                 pl.BlockSpec((BE, HALF), lambda i: (i, 0))],
      out_shape=[jax.ShapeDtypeStruct((E_EDGES, HALF), jnp.float32),
                 jax.ShapeDtypeStruct((E_EDGES, HALF), jnp.float32)],
  )(xb, w, b.reshape(1, HID))


def _rad_body(d_max, d2_ref, w_ref, b_ref, o0_ref, o1_ref):
  centers = (lax.broadcasted_iota(jnp.int32, (D_COUNT,), 0).astype(jnp.float32)
             * (d_max / (D_COUNT - 1)))
  width = d_max / D_COUNT
  d = jnp.sqrt(d2_ref[0, 0] + 1e-8)
  rbf = jnp.exp(-(((d[:, None] - centers[None, :]) / width) ** 2))
  z = jnp.dot(rbf, w_ref[...], preferred_element_type=jnp.float32) + b_ref[...]
  rad = _silu(z)
  o0_ref[...] = rad[:, :HALF]
  o1_ref[...] = rad[:, HALF:]


def _rad(d2, w, b, d_max):
  nb = E_EDGES // BE
  return pl.pallas_call(
      functools.partial(_rad_body, d_max),
      grid=(nb,),
      in_specs=[pl.BlockSpec((1, 1, BE), lambda i: (i, 0, 0)),
                pl.BlockSpec((D_COUNT, HID), lambda i: (0, 0)),
                pl.BlockSpec((1, HID), lambda i: (0, 0))],
      out_specs=[pl.BlockSpec((BE, HALF), lambda i: (i, 0)),
                 pl.BlockSpec((BE, HALF), lambda i: (i, 0))],
      out_shape=[jax.ShapeDtypeStruct((E_EDGES, HALF), jnp.float32),
                 jax.ShapeDtypeStruct((E_EDGES, HALF), jnp.float32)],
  )(d2.reshape(nb, 1, BE), w, b.reshape(1, HID))


BN_BLK = 400
BN_NB = N_NODES // BN_BLK


def _upd1_body(h0, h1, ac0, ac1, hba0, hba1, an0, an1, wc, bc, wn, bn,
               lc0, lc1, ln0, ln1, sc, qc, sn, qn):
  h = jnp.concatenate([h0[...], h1[...]], axis=1)
  ac = (jnp.concatenate([ac0[...], ac1[...]], axis=1) +
        jnp.concatenate([hba0[...], hba1[...]], axis=1))
  an = jnp.concatenate([an0[...], an1[...]], axis=1)
  zc = jnp.dot(h + ac, wc[...], preferred_element_type=jnp.float32) + bc[...]
  zc = jnp.where(zc >= 0, zc, 0.01 * zc)
  zn = jnp.dot(h + an, wn[...], preferred_element_type=jnp.float32) + bn[...]
  zn = jnp.where(zn >= 0, zn, 0.01 * zn)
  lc0[...] = zc[:, :HALF]
  lc1[...] = zc[:, HALF:]
  ln0[...] = zn[:, :HALF]
  ln1[...] = zn[:, HALF:]
  sc[...] = jnp.sum(zc, axis=0, keepdims=True)[None]
  qc[...] = jnp.sum(zc * zc, axis=0, keepdims=True)[None]
  sn[...] = jnp.sum(zn, axis=0, keepdims=True)[None]
  qn[...] = jnp.sum(zn * zn, axis=0, keepdims=True)[None]


def _upd1(h0, h1, ac0, ac1, hba0, hba1, an0, an1, wc, bc, wn, bn):
  half_spec = pl.BlockSpec((BN_BLK, HALF), lambda i: (i, 0))
  wspec = pl.BlockSpec((HID, HID), lambda i: (0, 0))
  bspec = pl.BlockSpec((1, HID), lambda i: (0, 0))
  pspec = pl.BlockSpec((1, 1, HID), lambda i: (i, 0, 0))
  return pl.pallas_call(
      _upd1_body,
      grid=(BN_NB,),
      in_specs=[half_spec] * 8 + [wspec, bspec, wspec, bspec],
      out_specs=[half_spec] * 4 + [pspec] * 4,
      out_shape=[jax.ShapeDtypeStruct((N_NODES, HALF), jnp.float32)] * 4 +
                [jax.ShapeDtypeStruct((BN_NB, 1, HID), jnp.float32)] * 4,
  )(h0, h1, ac0, ac1, hba0, hba1, an0, an1,
    wc, bc.reshape(1, HID), wn, bn.reshape(1, HID))


def _upd2_body(lc0, lc1, ln0, ln1, sc, qc, sn, qn, gc, betac, gn, betan,
               o0, o1):
  inv_n = 1.0 / N_NODES
  mc = jnp.sum(sc[...], axis=0) * inv_n
  vc = jnp.sum(qc[...], axis=0) * inv_n - mc * mc
  mn = jnp.sum(sn[...], axis=0) * inv_n
  vn = jnp.sum(qn[...], axis=0) * inv_n - mn * mn
  rc = jax.lax.rsqrt(vc + 1e-5)
  rn = jax.lax.rsqrt(vn + 1e-5)
  zc = jnp.concatenate([lc0[...], lc1[...]], axis=1)
  zn = jnp.concatenate([ln0[...], ln1[...]], axis=1)
  hc = (zc - mc) * rc * gc[...] + betac[...]
  hn = (zn - mn) * rn * gn[...] + betan[...]
  h = hc + hn
  o0[...] = h[:, :HALF]
  o1[...] = h[:, HALF:]


def _upd2(lc0, lc1, ln0, ln1, sc, qc, sn, qn, gc, betac, gn, betan):
  half_spec = pl.BlockSpec((BN_BLK, HALF), lambda i: (i, 0))
  pspec = pl.BlockSpec((BN_NB, 1, HID), lambda i: (0, 0, 0))
  bspec = pl.BlockSpec((1, HID), lambda i: (0, 0))
  return pl.pallas_call(
      _upd2_body,
      grid=(BN_NB,),
      in_specs=[half_spec] * 4 + [pspec] * 4 + [bspec] * 4,
      out_specs=[half_spec] * 2,
      out_shape=[jax.ShapeDtypeStruct((N_NODES, HALF), jnp.float32)] * 2,
  )(lc0, lc1, ln0, ln1, sc, qc, sn, qn,
    gc.reshape(1, HID), betac.reshape(1, HID),
    gn.reshape(1, HID), betan.reshape(1, HID))


def _head_body(h0, h1, batch, fw0, fb0, fg0, fbeta0, fw1, fb1, fg1, fbeta1,
               fw2, fb2, fg2, fbeta2, wout, bout, o_ref):
  h = jnp.concatenate([h0[...], h1[...]], axis=1)
  b = batch[...]
  gid = jax.lax.broadcasted_iota(jnp.int32, (N_NODES, N_GRAPHS), 1)
  onehot = (b == gid).astype(jnp.float32)
  emb = jax.lax.dot_general(onehot, h, (((0,), (0,)), ((), ())),
                            preferred_element_type=jnp.float32)
  z = emb
  for w, bb, g, beta in ((fw0, fb0, fg0, fbeta0), (fw1, fb1, fg1, fbeta1),
                         (fw2, fb2, fg2, fbeta2)):
    z = jnp.dot(z, w[...], preferred_element_type=jnp.float32) + bb[...]
    z = jnp.where(z >= 0, z, 0.01 * z)
    m = jnp.mean(z, axis=0, keepdims=True)
    v = jnp.mean(z * z, axis=0, keepdims=True) - m * m
    z = (z - m) * jax.lax.rsqrt(v + 1e-5) * g[...] + beta[...]
  out = jnp.dot(z, wout[...], preferred_element_type=jnp.float32) + bout[...]
  o_ref[...] = out.reshape(1, N_GRAPHS)


def _head(h0, h1, batch, fc, wout, bout):
  args = [h0, h1, batch.reshape(N_NODES, 1).astype(jnp.int32)]
  for p in fc:
    args += [p['W'], p['b'].reshape(1, HID), p['g'].reshape(1, HID),
             p['beta'].reshape(1, HID)]
  args += [wout, bout.reshape(1, 1)]
  return pl.pallas_call(
      _head_body,
      out_shape=jax.ShapeDtypeStruct((1, N_GRAPHS), jnp.float32),
  )(*args)


def kernel(x, x_bond, pos, params, edge_index_intra, edge_index_inter, batch):
  row_c = edge_index_intra[0].astype(jnp.int32)
  col_c = edge_index_intra[1].astype(jnp.int32)
  row_n = edge_index_inter[0].astype(jnp.int32)
  col_n = edge_index_inter[1].astype(jnp.int32)
  px = pos[:, 0].astype(jnp.float32)
  py = pos[:, 1].astype(jnp.float32)
  pz = pos[:, 2].astype(jnp.float32)

  d2_c, d2_n = _edge_d2(px, py, pz, row_c, col_c, row_n, col_n)

  h0, h1 = _node_embed(x, params['W_node'], params['b_node'])
  hb0, hb1 = _bond_embed(x_bond, params['W_bond'], params['b_bond'])
  hba0, hba1 = _hb_scatter(hb0, hb1, col_c)

  rads = [(_rad(d2_c, p['Wcc'], p['bcc'], 6.0),
           _rad(d2_n, p['Wcn'], p['bcn'], 10.0)) for p in params['gconv']]

  for p, ((radc0, radc1), (radn0, radn1)) in zip(params['gconv'], rads):
    ac0, ac1 = _gather_scatter(h0, h1, radc0, radc1, row_c, col_c)
    an0, an1 = _gather_scatter(h0, h1, radn0, radn1, row_n, col_n)
    lc0, lc1, ln0, ln1, sc, qc, sn, qn = _upd1(
        h0, h1, ac0, ac1, hba0, hba1, an0, an1,
        p['Wnc'], p['bnc'], p['Wnn'], p['bnn'])
    h0, h1 = _upd2(lc0, lc1, ln0, ln1, sc, qc, sn, qn,
                   p['gc'], p['betac'], p['gn'], p['betan'])

  out = _head(h0, h1, batch, params['fc'], params['W_out'], params['b_out'])
  return out.reshape(-1)


# speedup vs baseline: 3.2926x; 1.0015x over previous
import functools

import jax
import jax.numpy as jnp
from jax import lax
from jax.experimental import pallas as pl
from jax.experimental.pallas import tpu as pltpu
from jax.experimental.pallas import tpu_sc as plsc

N_NODES = 10000
E_EDGES = 160000
NODE_DIM = 35
BOND_DIM = 10
HID = 256
HALF = 128
D_COUNT = 64
N_GRAPHS = 64

NC = 2
NS = 16
L = 16


EPW = E_EDGES // (NC * NS)
_G = (EPW + L - 1) // L
_EPAD = _G * L


def _d2_body(px, py, pz, row_c, col_c, row_n, col_n, d2_c, d2_n,
             px_v, py_v, pz_v, ir_v, ic_v, out_v):
  c = lax.axis_index("c")
  s = lax.axis_index("s")
  w = s * NC + c
  pltpu.sync_copy(px, px_v)
  pltpu.sync_copy(py, py_v)
  pltpu.sync_copy(pz, pz_v)
  base = w * EPW
  for row_h, col_h, out_h in ((row_c, col_c, d2_c), (row_n, col_n, d2_n)):
    pltpu.sync_copy(row_h.at[pl.ds(base, EPW)], ir_v.at[pl.ds(0, EPW)])
    pltpu.sync_copy(col_h.at[pl.ds(base, EPW)], ic_v.at[pl.ds(0, EPW)])

    def grp(g, carry):
      sl = pl.ds(g * L, L)
      ri = jnp.clip(ir_v[sl], 0, N_NODES - 1)
      ci = jnp.clip(ic_v[sl], 0, N_NODES - 1)
      dx = plsc.load_gather(px_v, [ri]) - plsc.load_gather(px_v, [ci])
      dy = plsc.load_gather(py_v, [ri]) - plsc.load_gather(py_v, [ci])
      dz = plsc.load_gather(pz_v, [ri]) - plsc.load_gather(pz_v, [ci])
      out_v[sl] = dx * dx + dy * dy + dz * dz
      return carry

    lax.fori_loop(0, _G, grp, 0)
    pltpu.sync_copy(out_v.at[pl.ds(0, EPW)], out_h.at[pl.ds(base, EPW)])


def _edge_d2(px, py, pz, row_c, col_c, row_n, col_n):
  mesh = plsc.VectorSubcoreMesh(core_axis_name="c", subcore_axis_name="s",
                                num_cores=NC, num_subcores=NS)
  fn = pl.kernel(
      _d2_body,
      out_type=[jax.ShapeDtypeStruct((E_EDGES,), jnp.float32),
                jax.ShapeDtypeStruct((E_EDGES,), jnp.float32)],
      mesh=mesh,
      scratch_types=[
          pltpu.VMEM((N_NODES,), jnp.float32),
          pltpu.VMEM((N_NODES,), jnp.float32),
          pltpu.VMEM((N_NODES,), jnp.float32),
          pltpu.VMEM((_EPAD,), jnp.int32),
          pltpu.VMEM((_EPAD,), jnp.int32),
          pltpu.VMEM((_EPAD,), jnp.float32),
      ],
      compiler_params=pltpu.CompilerParams(needs_layout_passes=False),
  )
  return fn(px, py, pz, row_c, col_c, row_n, col_n)


KE = 40
EPT = E_EDGES // NS
NCHUNK = EPT // KE
N_PAD = 10240
RPT = N_PAD // NS
_ZREP = RPT // KE


def _zero_accum(accum, buf, s):
  def zrow(i, carry):
    for j in range(HALF // L):
      buf[i, pl.ds(j * L, L)] = jnp.zeros((L,), jnp.float32)
    return carry

  lax.fori_loop(0, KE, zrow, 0)
  base_row = s * RPT
  for t in range(_ZREP):
    pltpu.sync_copy(buf, accum.at[pl.ds(base_row + t * KE, KE)])


def _msg_body(h0, h1, rad0, rad1, row, col, agg0, agg1,
              accum, row_s, rows0, rows1, rows2, radv0, radv1, radv2,
              colv0, colv1, colv2, semd0, semd1, semd2, sems0, sems1, sems2):
  c = lax.axis_index("c")
  s = lax.axis_index("s")
  rows_r = (rows0, rows1, rows2)
  rad_r = (radv0, radv1, radv2)
  col_r = (colv0, colv1, colv2)
  semd = (semd0, semd1, semd2)
  sems = (sems0, sems1, sems2)
  _zero_accum(accum, rows0, s)
  plsc.subcore_barrier()

  ebase = s * EPT
  pltpu.sync_copy(row.at[pl.ds(ebase, EPT)], row_s)

  h_t = (h0, h1)
  rad_t = (rad0, rad1)

  for half in range(NC):

    @pl.when(c == half)
    def _():
      def descs(k, r):
        off = ebase + k * KE
        return [
            pltpu.make_async_copy(h_t[half].at[row_s.at[pl.ds(k * KE, KE)]],
                                  rows_r[r], semd[r]),
            pltpu.make_async_copy(
                rad_t[half].at[pl.ds(off * HALF, KE * HALF)],
                rad_r[r], semd[r]),
            pltpu.make_async_copy(col.at[pl.ds(off, KE)], col_r[r], semd[r]),
        ]

      def issue_load(k, r):
        for d in descs(k, r):
          d.start()

      def wait_load(k, r):
        for d in descs(k, r):
          d.wait()

      def mult(r):
        rows_v, rad_v = rows_r[r], rad_r[r]

        def mrow(i, carry2):
          for ii in range(2):
            r = 2 * i + ii
            for j in range(HALF // L):
              sl = pl.ds(j * L, L)
              rows_v[r, sl] = rows_v[r, sl] * rad_v[pl.ds(r * HALF + j * L, L)]
          return carry2

        lax.fori_loop(0, KE // 2, mrow, 0)

      def scatter_issue(r):
        pltpu.async_copy(rows_r[r], accum.at[col_r[r]], sems[r], add=True)

      def scatter_wait(r):
        pltpu.make_async_copy(rows_r[r], accum.at[col_r[r]], sems[r]).wait()

      issue_load(0, 0)
      wait_load(0, 0)
      issue_load(1, 1)
      mult(0)
      scatter_issue(0)
      wait_load(1, 1)
      issue_load(2, 2)

      def tri(t, carry):
        k = 3 * t + 1
        mult(1)
        scatter_issue(1)
        wait_load(k + 1, 2)
        scatter_wait(0)
        issue_load(k + 2, 0)
        mult(2)
        scatter_issue(2)
        scatter_wait(1)

        @pl.when(k + 3 < NCHUNK)
        def _():
          issue_load(k + 3, 1)

        wait_load(k + 2, 0)
        mult(0)
        scatter_issue(0)
        scatter_wait(2)

        @pl.when(k + 4 < NCHUNK)
        def _():
          issue_load(k + 4, 2)

        @pl.when(k + 3 < NCHUNK)
        def _():
          wait_load(k + 3, 1)

        return carry

      lax.fori_loop(0, (NCHUNK - 1) // 3, tri, 0)
      scatter_wait(0)

  plsc.subcore_barrier()
  base_row = s * RPT

  @pl.when(c == 0)
  def _():
    pltpu.sync_copy(accum.at[pl.ds(base_row, RPT)],
                    agg0.at[pl.ds(base_row, RPT)])

  @pl.when(c == 1)
  def _():
    pltpu.sync_copy(accum.at[pl.ds(base_row, RPT)],
                    agg1.at[pl.ds(base_row, RPT)])


def _gather_scatter(h0, h1, rad0, rad1, row, col):
  mesh = plsc.VectorSubcoreMesh(core_axis_name="c", subcore_axis_name="s",
                                num_cores=NC, num_subcores=NS)
  scratch = (
      [pltpu.VMEM_SHARED((N_PAD, HALF), jnp.float32),
       pltpu.VMEM((EPT,), jnp.int32)] +
      [pltpu.VMEM((KE, HALF), jnp.float32)] * 3 +
      [pltpu.VMEM((KE * HALF,), jnp.float32)] * 3 +
      [pltpu.VMEM((KE,), jnp.int32)] * 3 +
      [pltpu.SemaphoreType.DMA] * 6
  )
  fn = pl.kernel(
      _msg_body,
      out_type=[jax.ShapeDtypeStruct((N_PAD, HALF), jnp.float32),
                jax.ShapeDtypeStruct((N_PAD, HALF), jnp.float32)],
      mesh=mesh,
      scratch_types=scratch,
      compiler_params=pltpu.CompilerParams(needs_layout_passes=False),
  )
  a0, a1 = fn(h0, h1, rad0.reshape(-1), rad1.reshape(-1), row, col)
  return a0[:N_NODES], a1[:N_NODES]


def _hb_body(hb0, hb1, col, agg0, agg1,
             accum, buf_a, buf_b, col_a, col_b, sem):
  c = lax.axis_index("c")
  s = lax.axis_index("s")
  _zero_accum(accum, buf_a, s)
  plsc.subcore_barrier()
  hb_t = (hb0, hb1)
  ebase = s * EPT

  for half in range(NC):

    @pl.when(c == half)
    def _():
      def load(k, buf_v, col_v):
        off = ebase + k * KE
        return [pltpu.async_copy(hb_t[half].at[pl.ds(off, KE)], buf_v, sem),
                pltpu.async_copy(col.at[pl.ds(off, KE)], col_v, sem)]

      for d in load(0, buf_a, col_a):
        d.wait()

      def pair(g, carry):
        k = 2 * g
        da = load(k + 1, buf_b, col_b)
        pltpu.sync_copy(buf_a, accum.at[col_a], add=True)
        for d in da:
          d.wait()
        db = load(k + 2, buf_a, col_a)
        pltpu.sync_copy(buf_b, accum.at[col_b], add=True)
        for d in db:
          d.wait()
        return carry

      lax.fori_loop(0, NCHUNK // 2 - 1, pair, 0)
      da = load(NCHUNK - 1, buf_b, col_b)
      pltpu.sync_copy(buf_a, accum.at[col_a], add=True)
      for d in da:
        d.wait()
      pltpu.sync_copy(buf_b, accum.at[col_b], add=True)

  plsc.subcore_barrier()

  base_row = s * RPT

  @pl.when(c == 0)
  def _():
    pltpu.sync_copy(accum.at[pl.ds(base_row, RPT)],
                    agg0.at[pl.ds(base_row, RPT)])

  @pl.when(c == 1)
  def _():
    pltpu.sync_copy(accum.at[pl.ds(base_row, RPT)],
                    agg1.at[pl.ds(base_row, RPT)])


def _hb_scatter(hb0, hb1, col):
  mesh = plsc.VectorSubcoreMesh(core_axis_name="c", subcore_axis_name="s",
                                num_cores=NC, num_subcores=NS)
  fn = pl.kernel(
      _hb_body,
      out_type=[jax.ShapeDtypeStruct((N_PAD, HALF), jnp.float32),
                jax.ShapeDtypeStruct((N_PAD, HALF), jnp.float32)],
      mesh=mesh,
      scratch_types=[
          pltpu.VMEM_SHARED((N_PAD, HALF), jnp.float32),
          pltpu.VMEM((KE, HALF), jnp.float32),
          pltpu.VMEM((KE, HALF), jnp.float32),
          pltpu.VMEM((KE,), jnp.int32),
          pltpu.VMEM((KE,), jnp.int32),
          pltpu.SemaphoreType.DMA,
      ],
      compiler_params=pltpu.CompilerParams(needs_layout_passes=False),
  )
  a0, a1 = fn(hb0, hb1, col)
  return a0[:N_NODES], a1[:N_NODES]


def _silu(z):
  return z * jax.nn.sigmoid(z)


def _node_embed_body(x_ref, w_ref, b_ref, o0_ref, o1_ref):
  h = _silu(jnp.dot(x_ref[...], w_ref[...],
                    preferred_element_type=jnp.float32) + b_ref[...])
  o0_ref[...] = h[:, :HALF]
  o1_ref[...] = h[:, HALF:]


def _node_embed(x, w, b):
  return pl.pallas_call(
      _node_embed_body,
      out_shape=[jax.ShapeDtypeStruct((N_NODES, HALF), jnp.float32),
                 jax.ShapeDtypeStruct((N_NODES, HALF), jnp.float32)],
  )(x, w, b.reshape(1, HID))


BE = 2000


def _bond_embed_body(xb_ref, w_ref, b_ref, o0_ref, o1_ref):
  h = _silu(jnp.dot(xb_ref[...], w_ref[...],
                    preferred_element_type=jnp.float32) + b_ref[...])
  o0_ref[...] = h[:, :HALF]
  o1_ref[...] = h[:, HALF:]


def _bond_embed(xb, w, b):
  nb = E_EDGES // BE
  return pl.pallas_call(
      _bond_embed_body,
      grid=(nb,),
      in_specs=[pl.BlockSpec((BE, BOND_DIM), lambda i: (i, 0)),
                pl.BlockSpec((BOND_DIM, HID), lambda i: (0, 0)),
                pl.BlockSpec((1, HID), lambda i: (0, 0))],
      out_specs=[pl.BlockSpec((BE, HALF), lambda i: (i, 0)),
                 pl.BlockSpec((BE, HALF), lambda i: (i, 0))],
      out_shape=[jax.ShapeDtypeStruct((E_EDGES, HALF), jnp.float32),
                 jax.ShapeDtypeStruct((E_EDGES, HALF), jnp.float32)],
  )(xb, w, b.reshape(1, HID))


def _rad_body(d_max, d2_ref, w_ref, b_ref, o0_ref, o1_ref):
  centers = (lax.broadcasted_iota(jnp.int32, (D_COUNT,), 0).astype(jnp.float32)
             * (d_max / (D_COUNT - 1)))
  width = d_max / D_COUNT
  d = jnp.sqrt(d2_ref[0, 0] + 1e-8)
  rbf = jnp.exp(-(((d[:, None] - centers[None, :]) / width) ** 2))
  z = jnp.dot(rbf, w_ref[...], preferred_element_type=jnp.float32) + b_ref[...]
  rad = _silu(z)
  o0_ref[...] = rad[:, :HALF]
  o1_ref[...] = rad[:, HALF:]


_PERM = tuple(hh * HALF + g * 32 + off
              for hh in range(2) for g in range(4)
              for i in range(L) for off in (i, L + i))


def _rad(d2, w, b, d_max):
  nb = E_EDGES // BE
  return pl.pallas_call(
      functools.partial(_rad_body, d_max),
      grid=(nb,),
      in_specs=[pl.BlockSpec((1, 1, BE), lambda i: (i, 0, 0)),
                pl.BlockSpec((D_COUNT, HID), lambda i: (0, 0)),
                pl.BlockSpec((1, HID), lambda i: (0, 0))],
      out_specs=[pl.BlockSpec((BE, HALF), lambda i: (i, 0)),
                 pl.BlockSpec((BE, HALF), lambda i: (i, 0))],
      out_shape=[jax.ShapeDtypeStruct((E_EDGES, HALF), jnp.float32),
                 jax.ShapeDtypeStruct((E_EDGES, HALF), jnp.float32)],
  )(d2.reshape(nb, 1, BE), w, b.reshape(1, HID))


BN_BLK = 400
BN_NB = N_NODES // BN_BLK


def _upd1_body(h0, h1, ac0, ac1, hba0, hba1, an0, an1, wc, bc, wn, bn,
               lc0, lc1, ln0, ln1, sc, qc, sn, qn):
  h = jnp.concatenate([h0[...], h1[...]], axis=1)
  ac = (jnp.concatenate([ac0[...], ac1[...]], axis=1) +
        jnp.concatenate([hba0[...], hba1[...]], axis=1))
  an = jnp.concatenate([an0[...], an1[...]], axis=1)
  zc = jnp.dot(h + ac, wc[...], preferred_element_type=jnp.float32) + bc[...]
  zc = jnp.where(zc >= 0, zc, 0.01 * zc)
  zn = jnp.dot(h + an, wn[...], preferred_element_type=jnp.float32) + bn[...]
  zn = jnp.where(zn >= 0, zn, 0.01 * zn)
  lc0[...] = zc[:, :HALF]
  lc1[...] = zc[:, HALF:]
  ln0[...] = zn[:, :HALF]
  ln1[...] = zn[:, HALF:]
  sc[...] = jnp.sum(zc, axis=0, keepdims=True)[None]
  qc[...] = jnp.sum(zc * zc, axis=0, keepdims=True)[None]
  sn[...] = jnp.sum(zn, axis=0, keepdims=True)[None]
  qn[...] = jnp.sum(zn * zn, axis=0, keepdims=True)[None]


def _upd1(h0, h1, ac0, ac1, hba0, hba1, an0, an1, wc, bc, wn, bn):
  half_spec = pl.BlockSpec((BN_BLK, HALF), lambda i: (i, 0))
  wspec = pl.BlockSpec((HID, HID), lambda i: (0, 0))
  bspec = pl.BlockSpec((1, HID), lambda i: (0, 0))
  pspec = pl.BlockSpec((1, 1, HID), lambda i: (i, 0, 0))
  return pl.pallas_call(
      _upd1_body,
      grid=(BN_NB,),
      in_specs=[half_spec] * 8 + [wspec, bspec, wspec, bspec],
      out_specs=[half_spec] * 4 + [pspec] * 4,
      out_shape=[jax.ShapeDtypeStruct((N_NODES, HALF), jnp.float32)] * 4 +
                [jax.ShapeDtypeStruct((BN_NB, 1, HID), jnp.float32)] * 4,
  )(h0, h1, ac0, ac1, hba0, hba1, an0, an1,
    wc, bc.reshape(1, HID), wn, bn.reshape(1, HID))


def _upd2_body(lc0, lc1, ln0, ln1, sc, qc, sn, qn, gc, betac, gn, betan,
               o0, o1):
  inv_n = 1.0 / N_NODES
  mc = jnp.sum(sc[...], axis=0) * inv_n
  vc = jnp.sum(qc[...], axis=0) * inv_n - mc * mc
  mn = jnp.sum(sn[...], axis=0) * inv_n
  vn = jnp.sum(qn[...], axis=0) * inv_n - mn * mn
  rc = jax.lax.rsqrt(vc + 1e-5)
  rn = jax.lax.rsqrt(vn + 1e-5)
  zc = jnp.concatenate([lc0[...], lc1[...]], axis=1)
  zn = jnp.concatenate([ln0[...], ln1[...]], axis=1)
  hc = (zc - mc) * rc * gc[...] + betac[...]
  hn = (zn - mn) * rn * gn[...] + betan[...]
  h = hc + hn
  o0[...] = h[:, :HALF]
  o1[...] = h[:, HALF:]


def _upd2(lc0, lc1, ln0, ln1, sc, qc, sn, qn, gc, betac, gn, betan):
  half_spec = pl.BlockSpec((BN_BLK, HALF), lambda i: (i, 0))
  pspec = pl.BlockSpec((BN_NB, 1, HID), lambda i: (0, 0, 0))
  bspec = pl.BlockSpec((1, HID), lambda i: (0, 0))
  return pl.pallas_call(
      _upd2_body,
      grid=(BN_NB,),
      in_specs=[half_spec] * 4 + [pspec] * 4 + [bspec] * 4,
      out_specs=[half_spec] * 2,
      out_shape=[jax.ShapeDtypeStruct((N_NODES, HALF), jnp.float32)] * 2,
  )(lc0, lc1, ln0, ln1, sc, qc, sn, qn,
    gc.reshape(1, HID), betac.reshape(1, HID),
    gn.reshape(1, HID), betan.reshape(1, HID))


def _head_body(h0, h1, batch, fw0, fb0, fg0, fbeta0, fw1, fb1, fg1, fbeta1,
               fw2, fb2, fg2, fbeta2, wout, bout, o_ref):
  h = jnp.concatenate([h0[...], h1[...]], axis=1)
  b = batch[...]
  gid = jax.lax.broadcasted_iota(jnp.int32, (N_NODES, N_GRAPHS), 1)
  onehot = (b == gid).astype(jnp.float32)
  emb = jax.lax.dot_general(onehot, h, (((0,), (0,)), ((), ())),
                            preferred_element_type=jnp.float32)
  z = emb
  for w, bb, g, beta in ((fw0, fb0, fg0, fbeta0), (fw1, fb1, fg1, fbeta1),
                         (fw2, fb2, fg2, fbeta2)):
    z = jnp.dot(z, w[...], preferred_element_type=jnp.float32) + bb[...]
    z = jnp.where(z >= 0, z, 0.01 * z)
    m = jnp.mean(z, axis=0, keepdims=True)
    v = jnp.mean(z * z, axis=0, keepdims=True) - m * m
    z = (z - m) * jax.lax.rsqrt(v + 1e-5) * g[...] + beta[...]
  out = jnp.dot(z, wout[...], preferred_element_type=jnp.float32) + bout[...]
  o_ref[...] = out.reshape(1, N_GRAPHS)


def _head(h0, h1, batch, fc, wout, bout):
  args = [h0, h1, batch.reshape(N_NODES, 1).astype(jnp.int32)]
  for p in fc:
    args += [p['W'], p['b'].reshape(1, HID), p['g'].reshape(1, HID),
             p['beta'].reshape(1, HID)]
  args += [wout, bout.reshape(1, 1)]
  return pl.pallas_call(
      _head_body,
      out_shape=jax.ShapeDtypeStruct((1, N_GRAPHS), jnp.float32),
  )(*args)


def kernel(x, x_bond, pos, params, edge_index_intra, edge_index_inter, batch):
  row_c = edge_index_intra[0].astype(jnp.int32)
  col_c = edge_index_intra[1].astype(jnp.int32)
  row_n = edge_index_inter[0].astype(jnp.int32)
  col_n = edge_index_inter[1].astype(jnp.int32)
  px = pos[:, 0].astype(jnp.float32)
  py = pos[:, 1].astype(jnp.float32)
  pz = pos[:, 2].astype(jnp.float32)

  d2_c, d2_n = _edge_d2(px, py, pz, row_c, col_c, row_n, col_n)

  h0, h1 = _node_embed(x, params['W_node'], params['b_node'])
  hb0, hb1 = _bond_embed(x_bond, params['W_bond'], params['b_bond'])
  hba0, hba1 = _hb_scatter(hb0, hb1, col_c)

  rads = [(_rad(d2_c, p['Wcc'], p['bcc'], 6.0),
           _rad(d2_n, p['Wcn'], p['bcn'], 10.0)) for p in params['gconv']]

  for p, ((radc0, radc1), (radn0, radn1)) in zip(params['gconv'], rads):
    ac0, ac1 = _gather_scatter(h0, h1, radc0, radc1, row_c, col_c)
    an0, an1 = _gather_scatter(h0, h1, radn0, radn1, row_n, col_n)
    lc0, lc1, ln0, ln1, sc, qc, sn, qn = _upd1(
        h0, h1, ac0, ac1, hba0, hba1, an0, an1,
        p['Wnc'], p['bnc'], p['Wnn'], p['bnn'])
    h0, h1 = _upd2(lc0, lc1, ln0, ln1, sc, qc, sn, qn,
                   p['gc'], p['betac'], p['gn'], p['betan'])

  out = _head(h0, h1, batch, params['fc'], params['W_out'], params['b_out'])
  return out.reshape(-1)
